# baseline (speedup 1.0000x reference)
import functools

import jax
import jax.numpy as jnp
from jax import lax
from jax.experimental import pallas as pl
from jax.experimental.pallas import tpu as pltpu
from jax.experimental.pallas import tpu_sc as plsc

N = 10000
E = 320000
D = 128

NC = 2
NS = 16
NW = NC * NS
L = 16

CHUNK = 128
EPT = -(-E // (NW * CHUNK * 8)) * CHUNK * 8
E_PAD = EPT * NW
NCHUNK = EPT // CHUNK
N_PAD = 10240
QTR = N_PAD // 4
FL = N_PAD // NS
PFL = QTR // NS
BKT = EPT + 1024
BKTC = BKT // CHUNK
W = 64
NP = D // W

_mesh = plsc.VectorSubcoreMesh(core_axis_name="c", subcore_axis_name="s")
_sc_params = pltpu.CompilerParams(needs_layout_passes=False)


def _zero_1d(buf, n):
    def body(i, _):
        buf[pl.ds(i * L, L)] = jnp.zeros((L,), jnp.float32)
        return 0
    lax.fori_loop(0, n // L, body, 0)


@functools.partial(
    pl.kernel,
    out_type=[
        jax.ShapeDtypeStruct((NW * 4 * BKTC, CHUNK), jnp.int32),
        jax.ShapeDtypeStruct((NW * 4 * BKTC, CHUNK), jnp.int32),
        jax.ShapeDtypeStruct((NW * 4 * BKTC, CHUNK), jnp.int32),
        jax.ShapeDtypeStruct((NW * L,), jnp.int32),
    ],
    mesh=_mesh,
    compiler_params=_sc_params,
    scratch_types=[
        pltpu.VMEM((NCHUNK, CHUNK), jnp.int32),
        pltpu.VMEM((NCHUNK, CHUNK), jnp.int32),
        pltpu.VMEM((NCHUNK, CHUNK), jnp.float32),
        pltpu.VMEM((BKT,), jnp.int32),
        pltpu.VMEM((BKT,), jnp.int32),
        pltpu.VMEM((BKT,), jnp.int32),
        pltpu.VMEM((BKT,), jnp.int32),
        pltpu.VMEM((BKT,), jnp.float32),
        pltpu.VMEM((BKT,), jnp.float32),
        pltpu.VMEM((BKTC, CHUNK), jnp.int32),
        pltpu.VMEM((L,), jnp.int32),
    ],
)
def _part_kernel(src_hbm, dst_hbm, ew_hbm,
                 psrc_hbm, pdst_hbm, pew_hbm, pcnt_hbm,
                 srcs_v, dsts_v, ews_v,
                 bs0_v, bs1_v, bd0_v, bd1_v, bw0_v, bw1_v, d2_v, cnt_v):
    c = lax.axis_index("c")
    s = lax.axis_index("s")
    wid = c * NS + s

    pltpu.sync_copy(src_hbm.at[pl.ds(wid * NCHUNK, NCHUNK)], srcs_v)
    pltpu.sync_copy(dst_hbm.at[pl.ds(wid * NCHUNK, NCHUNK)], dsts_v)
    pltpu.sync_copy(ew_hbm.at[pl.ds(wid * NCHUNK, NCHUNK)], ews_v)

    zi = jnp.zeros((L,), jnp.int32)
    zf = jnp.zeros((L,), jnp.float32)

    def flush2d(flat, out2d, q, is_f32):
        def redist(r, _):
            for j in range(CHUNK // L):
                v = flat[pl.ds(r * CHUNK + j * L, L)]
                if is_f32:
                    v = plsc.bitcast(v, jnp.int32)
                d2_v[r, pl.ds(j * L, L)] = v
            return 0
        lax.fori_loop(0, BKTC, redist, 0)
        pltpu.sync_copy(d2_v, out2d.at[pl.ds((wid * 4 + q) * BKTC, BKTC)])

    qcnt = [None] * 4
    for r in (0, 1):
        lo = r * 2 * QTR

        def group_body(k, offs):
            off0, off1 = offs
            i = k // (CHUNK // L)
            g = k % (CHUNK // L)
            sl = pl.ds(g * L, L)
            s16 = srcs_v[i, sl]
            d16 = dsts_v[i, sl] - lo
            w16 = ews_v[i, sl]
            m0 = jnp.logical_and(d16 >= 0, d16 < QTR)
            m1 = jnp.logical_and(d16 >= QTR, d16 < 2 * QTR)
            plsc.store_compressed(bs0_v.at[pl.ds(off0, L)], s16, mask=m0)
            plsc.store_compressed(bd0_v.at[pl.ds(off0, L)], d16, mask=m0)
            plsc.store_compressed(bw0_v.at[pl.ds(off0, L)], w16, mask=m0)
            plsc.store_compressed(bs1_v.at[pl.ds(off1, L)], s16, mask=m1)
            plsc.store_compressed(bd1_v.at[pl.ds(off1, L)], d16 - QTR, mask=m1)
            plsc.store_compressed(bw1_v.at[pl.ds(off1, L)], w16, mask=m1)
            n0 = jnp.max(plsc.all_reduce_population_count(m0))
            n1 = jnp.max(plsc.all_reduce_population_count(m1))
            return (off0 + n0, off1 + n1)

        off0, off1 = lax.fori_loop(0, EPT // L, group_body, (0, 0))

        def pad_body(k, _):
            bs0_v[pl.ds(off0 + k * L, L)] = zi
            bd0_v[pl.ds(off0 + k * L, L)] = zi
            bw0_v[pl.ds(off0 + k * L, L)] = zf
            bs1_v[pl.ds(off1 + k * L, L)] = zi
            bd1_v[pl.ds(off1 + k * L, L)] = zi
            bw1_v[pl.ds(off1 + k * L, L)] = zf
            return 0
        lax.fori_loop(0, CHUNK // L, pad_body, 0)

        flush2d(bs0_v, psrc_hbm, 2 * r, False)
        flush2d(bs1_v, psrc_hbm, 2 * r + 1, False)
        flush2d(bd0_v, pdst_hbm, 2 * r, False)
        flush2d(bd1_v, pdst_hbm, 2 * r + 1, False)
        flush2d(bw0_v, pew_hbm, 2 * r, True)
        flush2d(bw1_v, pew_hbm, 2 * r + 1, True)
        qcnt[2 * r] = off0
        qcnt[2 * r + 1] = off1

    lane = lax.iota(jnp.int32, L)
    cv = jnp.zeros((L,), jnp.int32)
    for q in range(4):
        cv = jnp.where(lane == q, qcnt[q], cv)
    cnt_v[pl.ds(0, L)] = cv
    pltpu.sync_copy(cnt_v, pcnt_hbm.at[pl.ds(wid * L, L)])


@functools.partial(
    pl.kernel,
    out_type=jax.ShapeDtypeStruct((NW * N_PAD,), jnp.float32),
    mesh=_mesh,
    compiler_params=_sc_params,
    scratch_types=[
        pltpu.VMEM((NCHUNK, CHUNK), jnp.int32),
        pltpu.VMEM((NCHUNK, CHUNK), jnp.float32),
        pltpu.VMEM((N_PAD,), jnp.float32),
    ],
)
def _deg_kernel(dst_hbm, ew_hbm, out_hbm, dsts_v, ews_v, acc_v):
    c = lax.axis_index("c")
    s = lax.axis_index("s")
    wid = c * NS + s

    _zero_1d(acc_v, N_PAD)
    pltpu.sync_copy(dst_hbm.at[pl.ds(wid * NCHUNK, NCHUNK)], dsts_v)
    pltpu.sync_copy(ew_hbm.at[pl.ds(wid * NCHUNK, NCHUNK)], ews_v)

    def chunk_body(i, _):
        for g in range(CHUNK // L):
            sl = pl.ds(g * L, L)
            idx = dsts_v[i, sl]
            w = ews_v[i, sl]
            plsc.addupdate_scatter(acc_v, [idx], w)
        return 0

    lax.fori_loop(0, NCHUNK, chunk_body, 0)
    pltpu.sync_copy(acc_v, out_hbm.at[pl.ds(wid * N_PAD, N_PAD)])


@functools.partial(
    pl.kernel,
    out_type=jax.ShapeDtypeStruct((NP * N_PAD, W), jnp.float32),
    mesh=_mesh,
    compiler_params=pltpu.CompilerParams(
        needs_layout_passes=False, use_tc_tiling_on_sc=False),
    scratch_types=[
        pltpu.VMEM((BKTC, CHUNK), jnp.int32),
        pltpu.VMEM((BKTC, CHUNK), jnp.int32),
        pltpu.VMEM((BKTC, CHUNK), jnp.int32),
        pltpu.VMEM((BKTC, CHUNK), jnp.int32),
        pltpu.VMEM((BKTC, CHUNK), jnp.int32),
        pltpu.VMEM((BKTC, CHUNK), jnp.int32),
        pltpu.VMEM((CHUNK, W), jnp.float32),
        pltpu.VMEM((CHUNK, W), jnp.float32),
        pltpu.VMEM((PFL, W), jnp.float32),
        pltpu.VMEM((L,), jnp.int32),
        pltpu.VMEM_SHARED((QTR, W), jnp.float32),
        pltpu.SemaphoreType.DMA,
        pltpu.SemaphoreType.DMA,
        pltpu.SemaphoreType.DMA,
        pltpu.SemaphoreType.DMA,
    ],
)
def _prop_kernel(psrc_hbm, pdst_hbm, pew_hbm, pcnt_hbm,
                 g0_hbm, g1_hbm, out_hbm,
                 bsa_v, bda_v, bwa_v, bsb_v, bdb_v, bwb_v,
                 rows0_v, rows1_v, stage_v, cnt_v, acc_sh,
                 gsem0, gsem1, ssem0, ssem1):
    c = lax.axis_index("c")
    s = lax.axis_index("s")

    lane = lax.iota(jnp.int32, L)
    rows = (rows0_v, rows1_v)
    gsem = (gsem0, gsem1)
    ssem = (ssem0, ssem1)
    gp = (g0_hbm, g1_hbm)
    TSL = N_PAD // NS

    def bucket_cnt(t, q):
        pltpu.sync_copy(pcnt_hbm.at[pl.ds(t * L, L)], cnt_v)
        return jnp.max(jnp.where(lane == q, cnt_v[pl.ds(0, L)], 0))

    def load_bucket(t, q, bs, bd, bw):
        base = (t * 4 + q) * BKTC
        pltpu.sync_copy(psrc_hbm.at[pl.ds(base, BKTC)], bs)
        pltpu.sync_copy(pew_hbm.at[pl.ds(base, BKTC)], bw)
        pltpu.sync_copy(pdst_hbm.at[pl.ds(base, BKTC)], bd)

    def run_bucket(nch, tab, bs, bd, bw):
        def gather_start(i, b):
            pltpu.async_copy(tab.at[bs.at[i]], rows[b], gsem[b])

        def gather_wait(b):
            pltpu.make_async_copy(tab.at[bs.at[0]], rows[b], gsem[b]).wait()

        def scatter_start(i, b):
            pltpu.async_copy(rows[b], acc_sh.at[bd.at[i]], ssem[b], add=True)

        def scatter_wait(b):
            pltpu.make_async_copy(rows[b], acc_sh.at[bd.at[0]], ssem[b]).wait()

        pl.when(nch > 0)(lambda: gather_start(0, 0))

        def pair_body(i2, _):
            for b in (0, 1):
                i = 2 * i2 + b

                @pl.when(i < nch)
                def _process():
                    gather_wait(b)
                    pl.when(i >= 1)(lambda: scatter_wait(1 - b))
                    pl.when(i + 1 < nch)(lambda: gather_start(i + 1, 1 - b))

                    rv = rows[b]

                    def scale_body(e, _):
                        bc = plsc.bitcast(
                            plsc.load_gather(
                                bw,
                                [jnp.full((L,), i, jnp.int32),
                                 jnp.full((L,), e, jnp.int32)]),
                            jnp.float32)
                        for j in range(W // L):
                            sl = pl.ds(j * L, L)
                            rv[e, sl] = rv[e, sl] * bc
                        return 0
                    lax.fori_loop(0, CHUNK, scale_body, 0)

                    scatter_start(i, b)
            return 0

        lax.fori_loop(0, (nch + 1) // 2, pair_body, 0)
        pl.when(nch % 2 == 1)(lambda: scatter_wait(0))
        pl.when(jnp.logical_and(nch > 0, nch % 2 == 0))(
            lambda: scatter_wait(1))

    for p_outer in (0, 1):
        q = 2 * c + p_outer
        ncha = lax.div(bucket_cnt(2 * s, q) + (CHUNK - 1), CHUNK)
        nchb = lax.div(bucket_cnt(2 * s + 1, q) + (CHUNK - 1), CHUNK)
        load_bucket(2 * s, q, bsa_v, bda_v, bwa_v)
        load_bucket(2 * s + 1, q, bsb_v, bdb_v, bwb_v)

        for p in range(NP):
            def zero_body(i, _):
                for j in range(W // L):
                    stage_v[i, pl.ds(j * L, L)] = jnp.zeros((L,), jnp.float32)
                return 0
            lax.fori_loop(0, PFL, zero_body, 0)
            pltpu.sync_copy(stage_v, acc_sh.at[pl.ds(s * PFL, PFL)])
            plsc.subcore_barrier()

            run_bucket(ncha, gp[p], bsa_v, bda_v, bwa_v)
            run_bucket(nchb, gp[p], bsb_v, bdb_v, bwb_v)
            plsc.subcore_barrier()

            pltpu.sync_copy(acc_sh.at[pl.ds(s * PFL, PFL)], stage_v)
            pltpu.sync_copy(
                stage_v,
                out_hbm.at[pl.ds(p * N_PAD + q * QTR + s * PFL, PFL)])


@functools.partial(
    pl.kernel,
    out_type=jax.ShapeDtypeStruct((NW * N_PAD,), jnp.float32),
    mesh=_mesh,
    compiler_params=_sc_params,
    scratch_types=[
        pltpu.VMEM((NCHUNK, CHUNK), jnp.int32),
        pltpu.VMEM((NCHUNK, CHUNK), jnp.int32),
        pltpu.VMEM((NCHUNK, CHUNK), jnp.float32),
        pltpu.VMEM((N_PAD,), jnp.float32),
        pltpu.VMEM((N_PAD,), jnp.float32),
    ],
)
def _prop1_kernel(src_hbm, dst_hbm, ew_hbm, g_hbm, out_hbm,
                  srcs_v, dsts_v, ews_v, tab_v, acc_v):
    c = lax.axis_index("c")
    s = lax.axis_index("s")
    wid = c * NS + s

    _zero_1d(acc_v, N_PAD)
    pltpu.sync_copy(g_hbm, tab_v)
    pltpu.sync_copy(src_hbm.at[pl.ds(wid * NCHUNK, NCHUNK)], srcs_v)
    pltpu.sync_copy(dst_hbm.at[pl.ds(wid * NCHUNK, NCHUNK)], dsts_v)
    pltpu.sync_copy(ew_hbm.at[pl.ds(wid * NCHUNK, NCHUNK)], ews_v)

    def chunk_body(i, _):
        for g in range(CHUNK // L):
            sl = pl.ds(g * L, L)
            sidx = srcs_v[i, sl]
            didx = dsts_v[i, sl]
            w = ews_v[i, sl]
            vals = plsc.load_gather(tab_v, [sidx])
            plsc.addupdate_scatter(acc_v, [didx], vals * w)
        return 0

    lax.fori_loop(0, NCHUNK, chunk_body, 0)
    pltpu.sync_copy(acc_v, out_hbm.at[pl.ds(wid * N_PAD, N_PAD)])


_TB = 1024
_TGRID = N_PAD // _TB


def _plane_specs():
    return [pl.BlockSpec((_TB, W), lambda i: (i, 0)) for _ in range(NP)]


def _t1_body(p_ref, x_ref, w_ref, dinv_ref, *g_refs):
    deg = 1.0 + jnp.sum(p_ref[...], axis=1, keepdims=True)
    dinv = jnp.where(deg > 0, lax.rsqrt(deg), 0.0)
    dinv_ref[...] = dinv
    t = jnp.dot(x_ref[...], w_ref[...], preferred_element_type=jnp.float32)
    t = t * dinv
    for p in range(NP):
        g_refs[p][...] = t[:, p * W:(p + 1) * W]


def _tc_first(pdeg, x, W1):
    return pl.pallas_call(
        _t1_body,
        grid=(_TGRID,),
        in_specs=[
            pl.BlockSpec((_TB, NW), lambda i: (i, 0)),
            pl.BlockSpec((_TB, D), lambda i: (i, 0)),
            pl.BlockSpec((D, D), lambda i: (0, 0)),
        ],
        out_specs=[pl.BlockSpec((_TB, 1), lambda i: (i, 0))] + _plane_specs(),
        out_shape=[jax.ShapeDtypeStruct((N_PAD, 1), jnp.float32)]
        + [jax.ShapeDtypeStruct((N_PAD, W), jnp.float32) for _ in range(NP)],
    )(pdeg, x, W1)


def _t2_body(*refs, split_out):
    splanes = refs[:NP]
    gplanes = refs[NP:2 * NP]
    dinv_ref, b_ref, w_ref = refs[2 * NP:2 * NP + 3]
    outs = refs[2 * NP + 3:]
    dinv = dinv_ref[...]
    sfull = jnp.concatenate([r[...] for r in splanes], axis=1)
    gfull = jnp.concatenate([r[...] for r in gplanes], axis=1)
    h = dinv * (sfull + gfull) + b_ref[...]
    h = jnp.maximum(h, 0.0)
    t = jnp.dot(h, w_ref[...], preferred_element_type=jnp.float32)
    t = t * dinv
    if split_out:
        for p in range(NP):
            outs[p][...] = t[:, p * W:(p + 1) * W]
    else:
        outs[0][...] = t


def _tc_mid(splanes, gplanes, dinv, b, Wm, split_out):
    kout = Wm.shape[1]
    if split_out:
        out_specs = _plane_specs()
        out_shape = [jax.ShapeDtypeStruct((N_PAD, W), jnp.float32)
                     for _ in range(NP)]
    else:
        out_specs = [pl.BlockSpec((_TB, kout), lambda i: (i, 0))]
        out_shape = [jax.ShapeDtypeStruct((N_PAD, kout), jnp.float32)]
    return pl.pallas_call(
        functools.partial(_t2_body, split_out=split_out),
        grid=(_TGRID,),
        in_specs=_plane_specs() + _plane_specs() + [
            pl.BlockSpec((_TB, 1), lambda i: (i, 0)),
            pl.BlockSpec((1, D), lambda i: (0, 0)),
            pl.BlockSpec((D, kout), lambda i: (0, 0)),
        ],
        out_specs=out_specs,
        out_shape=out_shape,
    )(*splanes, *gplanes, dinv, b, Wm)


def _t4_body(p_ref, g_ref, dinv_ref, b_ref, out_ref):
    sarr = jnp.sum(p_ref[...], axis=1, keepdims=True)
    dinv = dinv_ref[...]
    h = dinv * (sarr + g_ref[...]) + b_ref[...]
    out_ref[...] = jnp.maximum(h, 0.0)


def _tc_final(p, g, dinv, b3):
    return pl.pallas_call(
        _t4_body,
        grid=(_TGRID,),
        in_specs=[
            pl.BlockSpec((_TB, NW), lambda i: (i, 0)),
            pl.BlockSpec((_TB, 1), lambda i: (i, 0)),
            pl.BlockSpec((_TB, 1), lambda i: (i, 0)),
            pl.BlockSpec((1, 1), lambda i: (0, 0)),
        ],
        out_specs=pl.BlockSpec((_TB, 1), lambda i: (i, 0)),
        out_shape=jax.ShapeDtypeStruct((N_PAD, 1), jnp.float32),
    )(p, g, dinv, b3)


@jax.jit
def kernel(x, edge_index, edge_attr, W1, b1, W2, b2, W3, b3):
    src = edge_index[0]
    dst = edge_index[1]
    ew = jnp.squeeze(edge_attr)

    pad = E_PAD - E
    src_p = jnp.concatenate([src, jnp.zeros((pad,), jnp.int32)])
    dst_p = jnp.concatenate([dst, jnp.zeros((pad,), jnp.int32)])
    ew_p = jnp.concatenate([ew, jnp.zeros((pad,), jnp.float32)])
    src2 = src_p.reshape(NW * NCHUNK, CHUNK)
    dst2 = dst_p.reshape(NW * NCHUNK, CHUNK)
    ew2 = ew_p.reshape(NW * NCHUNK, CHUNK)

    x_pad = jnp.pad(x, ((0, N_PAD - N), (0, 0)))

    psrc, pdst, pew, pcnt = _part_kernel(src2, dst2, ew2)

    degp = _deg_kernel(dst2, ew2).reshape(NW, N_PAD).T

    t1 = _tc_first(degp, x_pad, W1)
    dinv, g1p = t1[0], t1[1:]

    s1 = _prop_kernel(psrc, pdst, pew, pcnt, *g1p).reshape(NP, N_PAD, W)

    g2p = _tc_mid(list(s1), list(g1p), dinv, b1.reshape(1, D), W2, True)

    s2 = _prop_kernel(psrc, pdst, pew, pcnt, *g2p).reshape(NP, N_PAD, W)

    (g3,) = _tc_mid(list(s2), list(g2p), dinv, b2.reshape(1, D), W3, False)

    s3 = _prop1_kernel(src2, dst2, ew2, g3[:, 0]).reshape(NW, N_PAD).T

    out = _tc_final(s3, g3, dinv, b3.reshape(1, 1))
    return jnp.squeeze(out[:N])


# speedup vs baseline: 1.1018x; 1.1018x over previous
import functools

import jax
import jax.numpy as jnp
from jax import lax
from jax.experimental import pallas as pl
from jax.experimental.pallas import tpu as pltpu
from jax.experimental.pallas import tpu_sc as plsc

N = 10000
E = 320000
D = 128

NC = 2
NS = 16
NW = NC * NS
L = 16

CHUNK = 128
EPT = -(-E // (NW * CHUNK * 8)) * CHUNK * 8
E_PAD = EPT * NW
NCHUNK = EPT // CHUNK
N_PAD = 10240
QTR = N_PAD // 8
FL = N_PAD // NS
PFL = QTR // NS
BKT = EPT + 1024
BKTC = BKT // CHUNK
W = 32
NP = D // W

_mesh = plsc.VectorSubcoreMesh(core_axis_name="c", subcore_axis_name="s")
_sc_params = pltpu.CompilerParams(needs_layout_passes=False)


def _zero_1d(buf, n):
    def body(i, _):
        buf[pl.ds(i * L, L)] = jnp.zeros((L,), jnp.float32)
        return 0
    lax.fori_loop(0, n // L, body, 0)


@functools.partial(
    pl.kernel,
    out_type=[
        jax.ShapeDtypeStruct((NW * 8 * BKTC, CHUNK), jnp.int32),
        jax.ShapeDtypeStruct((NW * 8 * BKTC, CHUNK), jnp.int32),
        jax.ShapeDtypeStruct((NW * 8 * BKTC, CHUNK), jnp.int32),
        jax.ShapeDtypeStruct((NW * L,), jnp.int32),
    ],
    mesh=_mesh,
    compiler_params=_sc_params,
    scratch_types=[
        pltpu.VMEM((NCHUNK, CHUNK), jnp.int32),
        pltpu.VMEM((NCHUNK, CHUNK), jnp.int32),
        pltpu.VMEM((NCHUNK, CHUNK), jnp.float32),
        pltpu.VMEM((BKT,), jnp.int32),
        pltpu.VMEM((BKT,), jnp.int32),
        pltpu.VMEM((BKT,), jnp.int32),
        pltpu.VMEM((BKT,), jnp.int32),
        pltpu.VMEM((BKT,), jnp.float32),
        pltpu.VMEM((BKT,), jnp.float32),
        pltpu.VMEM((BKTC, CHUNK), jnp.int32),
        pltpu.VMEM((L,), jnp.int32),
    ],
)
def _part_kernel(src_hbm, dst_hbm, ew_hbm,
                 psrc_hbm, pdst_hbm, pew_hbm, pcnt_hbm,
                 srcs_v, dsts_v, ews_v,
                 bs0_v, bs1_v, bd0_v, bd1_v, bw0_v, bw1_v, d2_v, cnt_v):
    c = lax.axis_index("c")
    s = lax.axis_index("s")
    wid = c * NS + s

    pltpu.sync_copy(src_hbm.at[pl.ds(wid * NCHUNK, NCHUNK)], srcs_v)
    pltpu.sync_copy(dst_hbm.at[pl.ds(wid * NCHUNK, NCHUNK)], dsts_v)
    pltpu.sync_copy(ew_hbm.at[pl.ds(wid * NCHUNK, NCHUNK)], ews_v)

    zi = jnp.zeros((L,), jnp.int32)
    zf = jnp.zeros((L,), jnp.float32)

    def flush2d(flat, out2d, q, is_f32):
        def redist(r, _):
            for j in range(CHUNK // L):
                v = flat[pl.ds(r * CHUNK + j * L, L)]
                if is_f32:
                    v = plsc.bitcast(v, jnp.int32)
                d2_v[r, pl.ds(j * L, L)] = v
            return 0
        lax.fori_loop(0, BKTC, redist, 0)
        pltpu.sync_copy(d2_v, out2d.at[pl.ds((wid * 8 + q) * BKTC, BKTC)])

    qcnt = [None] * 8
    for r in range(4):
        lo = r * 2 * QTR

        def group_body(k, offs):
            off0, off1 = offs
            i = k // (CHUNK // L)
            g = k % (CHUNK // L)
            sl = pl.ds(g * L, L)
            s16 = srcs_v[i, sl]
            d16 = dsts_v[i, sl] - lo
            w16 = ews_v[i, sl]
            m0 = jnp.logical_and(d16 >= 0, d16 < QTR)
            m1 = jnp.logical_and(d16 >= QTR, d16 < 2 * QTR)
            plsc.store_compressed(bs0_v.at[pl.ds(off0, L)], s16, mask=m0)
            plsc.store_compressed(bd0_v.at[pl.ds(off0, L)], d16, mask=m0)
            plsc.store_compressed(bw0_v.at[pl.ds(off0, L)], w16, mask=m0)
            plsc.store_compressed(bs1_v.at[pl.ds(off1, L)], s16, mask=m1)
            plsc.store_compressed(bd1_v.at[pl.ds(off1, L)], d16 - QTR, mask=m1)
            plsc.store_compressed(bw1_v.at[pl.ds(off1, L)], w16, mask=m1)
            n0 = jnp.max(plsc.all_reduce_population_count(m0))
            n1 = jnp.max(plsc.all_reduce_population_count(m1))
            return (off0 + n0, off1 + n1)

        off0, off1 = lax.fori_loop(0, EPT // L, group_body, (0, 0))

        def pad_body(k, _):
            bs0_v[pl.ds(off0 + k * L, L)] = zi
            bd0_v[pl.ds(off0 + k * L, L)] = zi
            bw0_v[pl.ds(off0 + k * L, L)] = zf
            bs1_v[pl.ds(off1 + k * L, L)] = zi
            bd1_v[pl.ds(off1 + k * L, L)] = zi
            bw1_v[pl.ds(off1 + k * L, L)] = zf
            return 0
        lax.fori_loop(0, CHUNK // L, pad_body, 0)

        flush2d(bs0_v, psrc_hbm, 2 * r, False)
        flush2d(bs1_v, psrc_hbm, 2 * r + 1, False)
        flush2d(bd0_v, pdst_hbm, 2 * r, False)
        flush2d(bd1_v, pdst_hbm, 2 * r + 1, False)
        flush2d(bw0_v, pew_hbm, 2 * r, True)
        flush2d(bw1_v, pew_hbm, 2 * r + 1, True)
        qcnt[2 * r] = off0
        qcnt[2 * r + 1] = off1

    lane = lax.iota(jnp.int32, L)
    cv = jnp.zeros((L,), jnp.int32)
    for q in range(8):
        cv = jnp.where(lane == q, qcnt[q], cv)
    cnt_v[pl.ds(0, L)] = cv
    pltpu.sync_copy(cnt_v, pcnt_hbm.at[pl.ds(wid * L, L)])


@functools.partial(
    pl.kernel,
    out_type=jax.ShapeDtypeStruct((NW * N_PAD,), jnp.float32),
    mesh=_mesh,
    compiler_params=_sc_params,
    scratch_types=[
        pltpu.VMEM((NCHUNK, CHUNK), jnp.int32),
        pltpu.VMEM((NCHUNK, CHUNK), jnp.float32),
        pltpu.VMEM((N_PAD,), jnp.float32),
    ],
)
def _deg_kernel(dst_hbm, ew_hbm, out_hbm, dsts_v, ews_v, acc_v):
    c = lax.axis_index("c")
    s = lax.axis_index("s")
    wid = c * NS + s

    _zero_1d(acc_v, N_PAD)
    pltpu.sync_copy(dst_hbm.at[pl.ds(wid * NCHUNK, NCHUNK)], dsts_v)
    pltpu.sync_copy(ew_hbm.at[pl.ds(wid * NCHUNK, NCHUNK)], ews_v)

    def chunk_body(i, _):
        for g in range(CHUNK // L):
            sl = pl.ds(g * L, L)
            idx = dsts_v[i, sl]
            w = ews_v[i, sl]
            plsc.addupdate_scatter(acc_v, [idx], w)
        return 0

    lax.fori_loop(0, NCHUNK, chunk_body, 0)
    pltpu.sync_copy(acc_v, out_hbm.at[pl.ds(wid * N_PAD, N_PAD)])


@functools.partial(
    pl.kernel,
    out_type=jax.ShapeDtypeStruct((NP * N_PAD, W), jnp.float32),
    mesh=_mesh,
    compiler_params=pltpu.CompilerParams(
        needs_layout_passes=False, use_tc_tiling_on_sc=False),
    scratch_types=[
        pltpu.VMEM((BKTC, CHUNK), jnp.int32),
        pltpu.VMEM((BKTC, CHUNK), jnp.int32),
        pltpu.VMEM((BKTC, CHUNK), jnp.int32),
        pltpu.VMEM((BKTC, CHUNK), jnp.int32),
        pltpu.VMEM((BKTC, CHUNK), jnp.int32),
        pltpu.VMEM((BKTC, CHUNK), jnp.int32),
        pltpu.VMEM((CHUNK, W), jnp.float32),
        pltpu.VMEM((CHUNK, W), jnp.float32),
        pltpu.VMEM((PFL, W), jnp.float32),
        pltpu.VMEM((L,), jnp.int32),
        pltpu.VMEM_SHARED((N_PAD, W), jnp.float32),
        pltpu.VMEM_SHARED((QTR, W), jnp.float32),
        pltpu.SemaphoreType.DMA,
        pltpu.SemaphoreType.DMA,
        pltpu.SemaphoreType.DMA,
        pltpu.SemaphoreType.DMA,
    ],
)
def _prop_kernel(psrc_hbm, pdst_hbm, pew_hbm, pcnt_hbm,
                 g0_hbm, g1_hbm, g2_hbm, g3_hbm, out_hbm,
                 bsa_v, bda_v, bwa_v, bsb_v, bdb_v, bwb_v,
                 rows0_v, rows1_v, stage_v, cnt_v, tab_sh, acc_sh,
                 gsem0, gsem1, ssem0, ssem1):
    c = lax.axis_index("c")
    s = lax.axis_index("s")

    lane = lax.iota(jnp.int32, L)
    rows = (rows0_v, rows1_v)
    gsem = (gsem0, gsem1)
    ssem = (ssem0, ssem1)
    gp = (g0_hbm, g1_hbm, g2_hbm, g3_hbm)
    TSL = N_PAD // NS

    def bucket_cnt(t, q):
        pltpu.sync_copy(pcnt_hbm.at[pl.ds(t * L, L)], cnt_v)
        return jnp.max(jnp.where(lane == q, cnt_v[pl.ds(0, L)], 0))

    def load_bucket(t, q, bs, bd, bw):
        base = (t * 8 + q) * BKTC
        pltpu.sync_copy(psrc_hbm.at[pl.ds(base, BKTC)], bs)
        pltpu.sync_copy(pew_hbm.at[pl.ds(base, BKTC)], bw)
        pltpu.sync_copy(pdst_hbm.at[pl.ds(base, BKTC)], bd)

    def run_bucket(nch, bs, bd, bw):
        def gather_start(i, b):
            pltpu.async_copy(tab_sh.at[bs.at[i]], rows[b], gsem[b])

        def gather_wait(b):
            pltpu.make_async_copy(tab_sh.at[bs.at[0]], rows[b], gsem[b]).wait()

        def scatter_start(i, b):
            pltpu.async_copy(rows[b], acc_sh.at[bd.at[i]], ssem[b], add=True)

        def scatter_wait(b):
            pltpu.make_async_copy(rows[b], acc_sh.at[bd.at[0]], ssem[b]).wait()

        pl.when(nch > 0)(lambda: gather_start(0, 0))

        def pair_body(i2, _):
            for b in (0, 1):
                i = 2 * i2 + b

                @pl.when(i < nch)
                def _process():
                    gather_wait(b)
                    pl.when(i >= 1)(lambda: scatter_wait(1 - b))
                    pl.when(i + 1 < nch)(lambda: gather_start(i + 1, 1 - b))

                    rv = rows[b]

                    def scale_body(e, _):
                        bc = plsc.bitcast(
                            plsc.load_gather(
                                bw,
                                [jnp.full((L,), i, jnp.int32),
                                 jnp.full((L,), e, jnp.int32)]),
                            jnp.float32)
                        for j in range(W // L):
                            sl = pl.ds(j * L, L)
                            rv[e, sl] = rv[e, sl] * bc
                        return 0
                    lax.fori_loop(0, CHUNK, scale_body, 0)

                    scatter_start(i, b)
            return 0

        lax.fori_loop(0, (nch + 1) // 2, pair_body, 0)
        pl.when(nch % 2 == 1)(lambda: scatter_wait(0))
        pl.when(jnp.logical_and(nch > 0, nch % 2 == 0))(
            lambda: scatter_wait(1))

    def _run_segment(q, ncha, nchb):
        for p in range(NP):
            pltpu.sync_copy(gp[p].at[pl.ds(s * TSL, TSL)],
                            tab_sh.at[pl.ds(s * TSL, TSL)])

            def zero_body(i, _):
                for j in range(W // L):
                    stage_v[i, pl.ds(j * L, L)] = jnp.zeros((L,), jnp.float32)
                return 0
            lax.fori_loop(0, PFL, zero_body, 0)
            pltpu.sync_copy(stage_v, acc_sh.at[pl.ds(s * PFL, PFL)])
            plsc.subcore_barrier()

            run_bucket(ncha, bsa_v, bda_v, bwa_v)
            run_bucket(nchb, bsb_v, bdb_v, bwb_v)
            plsc.subcore_barrier()

            pltpu.sync_copy(acc_sh.at[pl.ds(s * PFL, PFL)], stage_v)
            pltpu.sync_copy(
                stage_v,
                out_hbm.at[pl.ds(p * N_PAD + q * QTR + s * PFL, PFL)])

    SEGS0 = (0, 1, 2, 3, 4)
    SEGS1 = (5, 6, 7, 0, 0)
    nseg = jnp.where(c == 0, 5, 3)
    for k in range(5):
        q = jnp.where(c == 0, SEGS0[k], SEGS1[k])

        seg_guard = pl.when(k < nseg)

        @seg_guard
        def _segment():
            ncha = lax.div(bucket_cnt(2 * s, q) + (CHUNK - 1), CHUNK)
            nchb = lax.div(bucket_cnt(2 * s + 1, q) + (CHUNK - 1), CHUNK)
            load_bucket(2 * s, q, bsa_v, bda_v, bwa_v)
            load_bucket(2 * s + 1, q, bsb_v, bdb_v, bwb_v)
            _run_segment(q, ncha, nchb)


@functools.partial(
    pl.kernel,
    out_type=jax.ShapeDtypeStruct((NW * N_PAD,), jnp.float32),
    mesh=_mesh,
    compiler_params=_sc_params,
    scratch_types=[
        pltpu.VMEM((NCHUNK, CHUNK), jnp.int32),
        pltpu.VMEM((NCHUNK, CHUNK), jnp.int32),
        pltpu.VMEM((NCHUNK, CHUNK), jnp.float32),
        pltpu.VMEM((N_PAD,), jnp.float32),
        pltpu.VMEM((N_PAD,), jnp.float32),
    ],
)
def _prop1_kernel(src_hbm, dst_hbm, ew_hbm, g_hbm, out_hbm,
                  srcs_v, dsts_v, ews_v, tab_v, acc_v):
    c = lax.axis_index("c")
    s = lax.axis_index("s")
    wid = c * NS + s

    _zero_1d(acc_v, N_PAD)
    pltpu.sync_copy(g_hbm, tab_v)
    pltpu.sync_copy(src_hbm.at[pl.ds(wid * NCHUNK, NCHUNK)], srcs_v)
    pltpu.sync_copy(dst_hbm.at[pl.ds(wid * NCHUNK, NCHUNK)], dsts_v)
    pltpu.sync_copy(ew_hbm.at[pl.ds(wid * NCHUNK, NCHUNK)], ews_v)

    def chunk_body(i, _):
        for g in range(CHUNK // L):
            sl = pl.ds(g * L, L)
            sidx = srcs_v[i, sl]
            didx = dsts_v[i, sl]
            w = ews_v[i, sl]
            vals = plsc.load_gather(tab_v, [sidx])
            plsc.addupdate_scatter(acc_v, [didx], vals * w)
        return 0

    lax.fori_loop(0, NCHUNK, chunk_body, 0)
    pltpu.sync_copy(acc_v, out_hbm.at[pl.ds(wid * N_PAD, N_PAD)])


_TB = 1024
_TGRID = N_PAD // _TB


def _plane_specs():
    return [pl.BlockSpec((_TB, W), lambda i: (i, 0)) for _ in range(NP)]


def _t1_body(p_ref, x_ref, w_ref, dinv_ref, *g_refs):
    deg = 1.0 + jnp.sum(p_ref[...], axis=1, keepdims=True)
    dinv = jnp.where(deg > 0, lax.rsqrt(deg), 0.0)
    dinv_ref[...] = dinv
    t = jnp.dot(x_ref[...], w_ref[...], preferred_element_type=jnp.float32)
    t = t * dinv
    for p in range(NP):
        g_refs[p][...] = t[:, p * W:(p + 1) * W]


def _tc_first(pdeg, x, W1):
    return pl.pallas_call(
        _t1_body,
        grid=(_TGRID,),
        in_specs=[
            pl.BlockSpec((_TB, NW), lambda i: (i, 0)),
            pl.BlockSpec((_TB, D), lambda i: (i, 0)),
            pl.BlockSpec((D, D), lambda i: (0, 0)),
        ],
        out_specs=[pl.BlockSpec((_TB, 1), lambda i: (i, 0))] + _plane_specs(),
        out_shape=[jax.ShapeDtypeStruct((N_PAD, 1), jnp.float32)]
        + [jax.ShapeDtypeStruct((N_PAD, W), jnp.float32) for _ in range(NP)],
    )(pdeg, x, W1)


def _t2_body(*refs, split_out):
    (s0, s1, s2, s3, g0, g1, g2, g3, dinv_ref, b_ref, w_ref) = refs[:11]
    outs = refs[11:]
    dinv = dinv_ref[...]
    sfull = jnp.concatenate([s0[...], s1[...], s2[...], s3[...]], axis=1)
    gfull = jnp.concatenate([g0[...], g1[...], g2[...], g3[...]], axis=1)
    h = dinv * (sfull + gfull) + b_ref[...]
    h = jnp.maximum(h, 0.0)
    t = jnp.dot(h, w_ref[...], preferred_element_type=jnp.float32)
    t = t * dinv
    if split_out:
        for p in range(NP):
            outs[p][...] = t[:, p * W:(p + 1) * W]
    else:
        outs[0][...] = t


def _tc_mid(splanes, gplanes, dinv, b, Wm, split_out):
    kout = Wm.shape[1]
    if split_out:
        out_specs = _plane_specs()
        out_shape = [jax.ShapeDtypeStruct((N_PAD, W), jnp.float32)
                     for _ in range(NP)]
    else:
        out_specs = [pl.BlockSpec((_TB, kout), lambda i: (i, 0))]
        out_shape = [jax.ShapeDtypeStruct((N_PAD, kout), jnp.float32)]
    return pl.pallas_call(
        functools.partial(_t2_body, split_out=split_out),
        grid=(_TGRID,),
        in_specs=_plane_specs() + _plane_specs() + [
            pl.BlockSpec((_TB, 1), lambda i: (i, 0)),
            pl.BlockSpec((1, D), lambda i: (0, 0)),
            pl.BlockSpec((D, kout), lambda i: (0, 0)),
        ],
        out_specs=out_specs,
        out_shape=out_shape,
    )(*splanes, *gplanes, dinv, b, Wm)


def _t4_body(p_ref, g_ref, dinv_ref, b_ref, out_ref):
    sarr = jnp.sum(p_ref[...], axis=1, keepdims=True)
    dinv = dinv_ref[...]
    h = dinv * (sarr + g_ref[...]) + b_ref[...]
    out_ref[...] = jnp.maximum(h, 0.0)


def _tc_final(p, g, dinv, b3):
    return pl.pallas_call(
        _t4_body,
        grid=(_TGRID,),
        in_specs=[
            pl.BlockSpec((_TB, NW), lambda i: (i, 0)),
            pl.BlockSpec((_TB, 1), lambda i: (i, 0)),
            pl.BlockSpec((_TB, 1), lambda i: (i, 0)),
            pl.BlockSpec((1, 1), lambda i: (0, 0)),
        ],
        out_specs=pl.BlockSpec((_TB, 1), lambda i: (i, 0)),
        out_shape=jax.ShapeDtypeStruct((N_PAD, 1), jnp.float32),
    )(p, g, dinv, b3)


@jax.jit
def kernel(x, edge_index, edge_attr, W1, b1, W2, b2, W3, b3):
    src = edge_index[0]
    dst = edge_index[1]
    ew = jnp.squeeze(edge_attr)

    pad = E_PAD - E
    src_p = jnp.concatenate([src, jnp.zeros((pad,), jnp.int32)])
    dst_p = jnp.concatenate([dst, jnp.zeros((pad,), jnp.int32)])
    ew_p = jnp.concatenate([ew, jnp.zeros((pad,), jnp.float32)])
    src2 = src_p.reshape(NW * NCHUNK, CHUNK)
    dst2 = dst_p.reshape(NW * NCHUNK, CHUNK)
    ew2 = ew_p.reshape(NW * NCHUNK, CHUNK)

    x_pad = jnp.pad(x, ((0, N_PAD - N), (0, 0)))

    psrc, pdst, pew, pcnt = _part_kernel(src2, dst2, ew2)

    degp = _deg_kernel(dst2, ew2).reshape(NW, N_PAD).T

    t1 = _tc_first(degp, x_pad, W1)
    dinv, g1p = t1[0], t1[1:]

    s1 = _prop_kernel(psrc, pdst, pew, pcnt, *g1p).reshape(NP, N_PAD, W)

    g2p = _tc_mid(list(s1), list(g1p), dinv, b1.reshape(1, D), W2, True)

    s2 = _prop_kernel(psrc, pdst, pew, pcnt, *g2p).reshape(NP, N_PAD, W)

    (g3,) = _tc_mid(list(s2), list(g2p), dinv, b2.reshape(1, D), W3, False)

    s3 = _prop1_kernel(src2, dst2, ew2, g3[:, 0]).reshape(NW, N_PAD).T

    out = _tc_final(s3, g3, dinv, b3.reshape(1, 1))
    return jnp.squeeze(out[:N])


# speedup vs baseline: 1.4256x; 1.2939x over previous
import functools

import jax
import jax.numpy as jnp
from jax import lax
from jax.experimental import pallas as pl
from jax.experimental.pallas import tpu as pltpu
from jax.experimental.pallas import tpu_sc as plsc

N = 10000
E = 320000
D = 128

NC = 2
NS = 16
NW = NC * NS
L = 16

CHUNK = 128
EPT = -(-E // (NW * CHUNK * 8)) * CHUNK * 8
E_PAD = EPT * NW
NCHUNK = EPT // CHUNK
N_PAD = 10240
QTR = N_PAD // 8
FL = N_PAD // NS
PFL = QTR // NS
BKT = EPT + 1024
BKTC = BKT // CHUNK
W = 32
NP = D // W

_mesh = plsc.VectorSubcoreMesh(core_axis_name="c", subcore_axis_name="s")
_sc_params = pltpu.CompilerParams(needs_layout_passes=False)


def _zero_1d(buf, n):
    def body(i, _):
        buf[pl.ds(i * L, L)] = jnp.zeros((L,), jnp.float32)
        return 0
    lax.fori_loop(0, n // L, body, 0)


@functools.partial(
    pl.kernel,
    out_type=[
        jax.ShapeDtypeStruct((NW * 8 * BKTC, CHUNK), jnp.int32),
        jax.ShapeDtypeStruct((NW * 8 * BKTC, CHUNK), jnp.int32),
        jax.ShapeDtypeStruct((NW * 8 * BKTC, CHUNK), jnp.int32),
        jax.ShapeDtypeStruct((NW * L,), jnp.int32),
    ],
    mesh=_mesh,
    compiler_params=_sc_params,
    scratch_types=[
        pltpu.VMEM((NCHUNK, CHUNK), jnp.int32),
        pltpu.VMEM((NCHUNK, CHUNK), jnp.int32),
        pltpu.VMEM((NCHUNK, CHUNK), jnp.float32),
        pltpu.VMEM((BKT,), jnp.int32),
        pltpu.VMEM((BKT,), jnp.int32),
        pltpu.VMEM((BKT,), jnp.int32),
        pltpu.VMEM((BKT,), jnp.int32),
        pltpu.VMEM((BKT,), jnp.float32),
        pltpu.VMEM((BKT,), jnp.float32),
        pltpu.VMEM((BKTC, CHUNK), jnp.int32),
        pltpu.VMEM((L,), jnp.int32),
    ],
)
def _part_kernel(src_hbm, dst_hbm, ew_hbm,
                 psrc_hbm, pdst_hbm, pew_hbm, pcnt_hbm,
                 srcs_v, dsts_v, ews_v,
                 bs0_v, bs1_v, bd0_v, bd1_v, bw0_v, bw1_v, d2_v, cnt_v):
    c = lax.axis_index("c")
    s = lax.axis_index("s")
    wid = c * NS + s

    pltpu.sync_copy(src_hbm.at[pl.ds(wid * NCHUNK, NCHUNK)], srcs_v)
    pltpu.sync_copy(dst_hbm.at[pl.ds(wid * NCHUNK, NCHUNK)], dsts_v)
    pltpu.sync_copy(ew_hbm.at[pl.ds(wid * NCHUNK, NCHUNK)], ews_v)

    zi = jnp.zeros((L,), jnp.int32)
    zf = jnp.zeros((L,), jnp.float32)

    def flush2d(flat, out2d, q, is_f32):
        def redist(r, _):
            for j in range(CHUNK // L):
                v = flat[pl.ds(r * CHUNK + j * L, L)]
                if is_f32:
                    v = plsc.bitcast(v, jnp.int32)
                d2_v[r, pl.ds(j * L, L)] = v
            return 0
        lax.fori_loop(0, BKTC, redist, 0)
        pltpu.sync_copy(d2_v, out2d.at[pl.ds((wid * 8 + q) * BKTC, BKTC)])

    qcnt = [None] * 8
    for r in range(4):
        lo = r * 2 * QTR

        def group_body(k, offs):
            off0, off1 = offs
            i = k // (CHUNK // L)
            g = k % (CHUNK // L)
            sl = pl.ds(g * L, L)
            s16 = srcs_v[i, sl]
            d16 = dsts_v[i, sl] - lo
            w16 = ews_v[i, sl]
            m0 = jnp.logical_and(d16 >= 0, d16 < QTR)
            m1 = jnp.logical_and(d16 >= QTR, d16 < 2 * QTR)
            plsc.store_compressed(bs0_v.at[pl.ds(off0, L)], s16, mask=m0)
            plsc.store_compressed(bd0_v.at[pl.ds(off0, L)], d16, mask=m0)
            plsc.store_compressed(bw0_v.at[pl.ds(off0, L)], w16, mask=m0)
            plsc.store_compressed(bs1_v.at[pl.ds(off1, L)], s16, mask=m1)
            plsc.store_compressed(bd1_v.at[pl.ds(off1, L)], d16 - QTR, mask=m1)
            plsc.store_compressed(bw1_v.at[pl.ds(off1, L)], w16, mask=m1)
            n0 = jnp.max(plsc.all_reduce_population_count(m0))
            n1 = jnp.max(plsc.all_reduce_population_count(m1))
            return (off0 + n0, off1 + n1)

        off0, off1 = lax.fori_loop(0, EPT // L, group_body, (0, 0))

        def pad_body(k, _):
            bs0_v[pl.ds(off0 + k * L, L)] = zi
            bd0_v[pl.ds(off0 + k * L, L)] = zi
            bw0_v[pl.ds(off0 + k * L, L)] = zf
            bs1_v[pl.ds(off1 + k * L, L)] = zi
            bd1_v[pl.ds(off1 + k * L, L)] = zi
            bw1_v[pl.ds(off1 + k * L, L)] = zf
            return 0
        lax.fori_loop(0, CHUNK // L, pad_body, 0)

        flush2d(bs0_v, psrc_hbm, 2 * r, False)
        flush2d(bs1_v, psrc_hbm, 2 * r + 1, False)
        flush2d(bd0_v, pdst_hbm, 2 * r, False)
        flush2d(bd1_v, pdst_hbm, 2 * r + 1, False)
        flush2d(bw0_v, pew_hbm, 2 * r, True)
        flush2d(bw1_v, pew_hbm, 2 * r + 1, True)
        qcnt[2 * r] = off0
        qcnt[2 * r + 1] = off1

    lane = lax.iota(jnp.int32, L)
    cv = jnp.zeros((L,), jnp.int32)
    for q in range(8):
        cv = jnp.where(lane == q, qcnt[q], cv)
    cnt_v[pl.ds(0, L)] = cv
    pltpu.sync_copy(cnt_v, pcnt_hbm.at[pl.ds(wid * L, L)])


@functools.partial(
    pl.kernel,
    out_type=jax.ShapeDtypeStruct((NW * N_PAD,), jnp.float32),
    mesh=_mesh,
    compiler_params=_sc_params,
    scratch_types=[
        pltpu.VMEM((NCHUNK, CHUNK), jnp.int32),
        pltpu.VMEM((NCHUNK, CHUNK), jnp.float32),
        pltpu.VMEM((N_PAD,), jnp.float32),
    ],
)
def _deg_kernel(dst_hbm, ew_hbm, out_hbm, dsts_v, ews_v, acc_v):
    c = lax.axis_index("c")
    s = lax.axis_index("s")
    wid = c * NS + s

    _zero_1d(acc_v, N_PAD)
    pltpu.sync_copy(dst_hbm.at[pl.ds(wid * NCHUNK, NCHUNK)], dsts_v)
    pltpu.sync_copy(ew_hbm.at[pl.ds(wid * NCHUNK, NCHUNK)], ews_v)

    def chunk_body(i, _):
        for g in range(CHUNK // L):
            sl = pl.ds(g * L, L)
            idx = dsts_v[i, sl]
            w = ews_v[i, sl]
            plsc.addupdate_scatter(acc_v, [idx], w)
        return 0

    lax.fori_loop(0, NCHUNK, chunk_body, 0)
    pltpu.sync_copy(acc_v, out_hbm.at[pl.ds(wid * N_PAD, N_PAD)])


@functools.partial(
    pl.kernel,
    out_type=jax.ShapeDtypeStruct((NP * N_PAD, W), jnp.float32),
    mesh=_mesh,
    compiler_params=pltpu.CompilerParams(
        needs_layout_passes=False, use_tc_tiling_on_sc=False),
    scratch_types=[
        pltpu.VMEM((BKTC, CHUNK), jnp.int32),
        pltpu.VMEM((BKTC, CHUNK), jnp.int32),
        pltpu.VMEM((BKTC, CHUNK), jnp.int32),
        pltpu.VMEM((BKTC, CHUNK), jnp.int32),
        pltpu.VMEM((BKTC, CHUNK), jnp.int32),
        pltpu.VMEM((BKTC, CHUNK), jnp.int32),
        pltpu.VMEM((CHUNK, W), jnp.float32),
        pltpu.VMEM((CHUNK, W), jnp.float32),
        pltpu.VMEM((PFL, W), jnp.float32),
        pltpu.VMEM((L,), jnp.int32),
        pltpu.VMEM_SHARED((N_PAD, W), jnp.float32),
        pltpu.VMEM_SHARED((QTR, W), jnp.float32),
        pltpu.SemaphoreType.DMA,
        pltpu.SemaphoreType.DMA,
        pltpu.SemaphoreType.DMA,
        pltpu.SemaphoreType.DMA,
    ],
)
def _prop_kernel(psrc_hbm, pdst_hbm, pew_hbm, pcnt_hbm,
                 g0_hbm, g1_hbm, g2_hbm, g3_hbm, out_hbm,
                 bsa_v, bda_v, bwa_v, bsb_v, bdb_v, bwb_v,
                 rows0_v, rows1_v, stage_v, cnt_v, tab_sh, acc_sh,
                 gsem0, gsem1, ssem0, ssem1):
    c = lax.axis_index("c")
    s = lax.axis_index("s")

    lane = lax.iota(jnp.int32, L)
    rows = (rows0_v, rows1_v)
    gsem = (gsem0, gsem1)
    ssem = (ssem0, ssem1)
    gp = (g0_hbm, g1_hbm, g2_hbm, g3_hbm)
    TSL = N_PAD // NS

    def bucket_cnt(t, q):
        pltpu.sync_copy(pcnt_hbm.at[pl.ds(t * L, L)], cnt_v)
        return jnp.max(jnp.where(lane == q, cnt_v[pl.ds(0, L)], 0))

    def load_bucket(t, q, bs, bd, bw):
        base = (t * 8 + q) * BKTC
        pltpu.sync_copy(psrc_hbm.at[pl.ds(base, BKTC)], bs)
        pltpu.sync_copy(pew_hbm.at[pl.ds(base, BKTC)], bw)
        pltpu.sync_copy(pdst_hbm.at[pl.ds(base, BKTC)], bd)

    def run_bucket(nch, bs, bd, bw):
        def gather_start(i, b):
            pltpu.async_copy(tab_sh.at[bs.at[i]], rows[b], gsem[b])

        def gather_wait(b):
            pltpu.make_async_copy(tab_sh.at[bs.at[0]], rows[b], gsem[b]).wait()

        def scatter_start(i, b):
            pltpu.async_copy(rows[b], acc_sh.at[bd.at[i]], ssem[b], add=True)

        def scatter_wait(b):
            pltpu.make_async_copy(rows[b], acc_sh.at[bd.at[0]], ssem[b]).wait()

        pl.when(nch > 0)(lambda: gather_start(0, 0))

        def pair_body(i2, _):
            for b in (0, 1):
                i = 2 * i2 + b

                @pl.when(i < nch)
                def _process():
                    gather_wait(b)
                    pl.when(i >= 1)(lambda: scatter_wait(1 - b))
                    pl.when(i + 1 < nch)(lambda: gather_start(i + 1, 1 - b))

                    rv = rows[b]

                    def scale_body(e, _):
                        bc = plsc.bitcast(
                            plsc.load_gather(
                                bw,
                                [jnp.full((L,), i, jnp.int32),
                                 jnp.full((L,), e, jnp.int32)]),
                            jnp.float32)
                        for j in range(W // L):
                            sl = pl.ds(j * L, L)
                            rv[e, sl] = rv[e, sl] * bc
                        return 0
                    lax.fori_loop(0, CHUNK, scale_body, 0)

                    scatter_start(i, b)
            return 0

        lax.fori_loop(0, (nch + 1) // 2, pair_body, 0)
        pl.when(nch % 2 == 1)(lambda: scatter_wait(0))
        pl.when(jnp.logical_and(nch > 0, nch % 2 == 0))(
            lambda: scatter_wait(1))

    def _run_segment(q, ncha, nchb):
        for p in range(NP):
            pltpu.sync_copy(gp[p].at[pl.ds(s * TSL, TSL)],
                            tab_sh.at[pl.ds(s * TSL, TSL)])

            def zero_body(i, _):
                for j in range(W // L):
                    stage_v[i, pl.ds(j * L, L)] = jnp.zeros((L,), jnp.float32)
                return 0
            lax.fori_loop(0, PFL, zero_body, 0)
            pltpu.sync_copy(stage_v, acc_sh.at[pl.ds(s * PFL, PFL)])
            plsc.subcore_barrier()

            run_bucket(ncha, bsa_v, bda_v, bwa_v)
            run_bucket(nchb, bsb_v, bdb_v, bwb_v)
            plsc.subcore_barrier()

            pltpu.sync_copy(acc_sh.at[pl.ds(s * PFL, PFL)], stage_v)
            pltpu.sync_copy(
                stage_v,
                out_hbm.at[pl.ds(p * N_PAD + q * QTR + s * PFL, PFL)])

    SEGS0 = (0, 1, 2, 0, 0)
    SEGS1 = (3, 4, 5, 6, 7)
    nseg = jnp.where(c == 0, 3, 5)
    for k in range(5):
        q = jnp.where(c == 0, SEGS0[k], SEGS1[k])

        seg_guard = pl.when(k < nseg)

        @seg_guard
        def _segment():
            ncha = lax.div(bucket_cnt(2 * s, q) + (CHUNK - 1), CHUNK)
            nchb = lax.div(bucket_cnt(2 * s + 1, q) + (CHUNK - 1), CHUNK)
            load_bucket(2 * s, q, bsa_v, bda_v, bwa_v)
            load_bucket(2 * s + 1, q, bsb_v, bdb_v, bwb_v)
            _run_segment(q, ncha, nchb)


@functools.partial(
    pl.kernel,
    out_type=jax.ShapeDtypeStruct((NW * N_PAD,), jnp.float32),
    mesh=_mesh,
    compiler_params=_sc_params,
    scratch_types=[
        pltpu.VMEM((NCHUNK, CHUNK), jnp.int32),
        pltpu.VMEM((NCHUNK, CHUNK), jnp.int32),
        pltpu.VMEM((NCHUNK, CHUNK), jnp.float32),
        pltpu.VMEM((N_PAD,), jnp.float32),
        pltpu.VMEM((N_PAD,), jnp.float32),
    ],
)
def _prop1_kernel(src_hbm, dst_hbm, ew_hbm, g_hbm, out_hbm,
                  srcs_v, dsts_v, ews_v, tab_v, acc_v):
    c = lax.axis_index("c")
    s = lax.axis_index("s")
    wid = c * NS + s

    _zero_1d(acc_v, N_PAD)
    pltpu.sync_copy(g_hbm, tab_v)
    pltpu.sync_copy(src_hbm.at[pl.ds(wid * NCHUNK, NCHUNK)], srcs_v)
    pltpu.sync_copy(dst_hbm.at[pl.ds(wid * NCHUNK, NCHUNK)], dsts_v)
    pltpu.sync_copy(ew_hbm.at[pl.ds(wid * NCHUNK, NCHUNK)], ews_v)

    def chunk_body(i, _):
        for g in range(CHUNK // L):
            sl = pl.ds(g * L, L)
            sidx = srcs_v[i, sl]
            didx = dsts_v[i, sl]
            w = ews_v[i, sl]
            vals = plsc.load_gather(tab_v, [sidx])
            plsc.addupdate_scatter(acc_v, [didx], vals * w)
        return 0

    lax.fori_loop(0, NCHUNK, chunk_body, 0)
    pltpu.sync_copy(acc_v, out_hbm.at[pl.ds(wid * N_PAD, N_PAD)])


_TB = 1024
_TGRID = N_PAD // _TB


def _plane_specs():
    return [pl.BlockSpec((_TB, W), lambda i: (i, 0)) for _ in range(NP)]


def _t1_body(p_ref, x_ref, w_ref, dinv_ref, *g_refs):
    deg = 1.0 + jnp.sum(p_ref[...], axis=1, keepdims=True)
    dinv = jnp.where(deg > 0, lax.rsqrt(deg), 0.0)
    dinv_ref[...] = dinv
    t = jnp.dot(x_ref[...], w_ref[...], preferred_element_type=jnp.float32)
    t = t * dinv
    for p in range(NP):
        g_refs[p][...] = t[:, p * W:(p + 1) * W]


def _tc_first(pdeg, x, W1):
    return pl.pallas_call(
        _t1_body,
        grid=(_TGRID,),
        in_specs=[
            pl.BlockSpec((_TB, NW), lambda i: (i, 0)),
            pl.BlockSpec((_TB, D), lambda i: (i, 0)),
            pl.BlockSpec((D, D), lambda i: (0, 0)),
        ],
        out_specs=[pl.BlockSpec((_TB, 1), lambda i: (i, 0))] + _plane_specs(),
        out_shape=[jax.ShapeDtypeStruct((N_PAD, 1), jnp.float32)]
        + [jax.ShapeDtypeStruct((N_PAD, W), jnp.float32) for _ in range(NP)],
    )(pdeg, x, W1)


def _t2_body(*refs, split_out):
    (s0, s1, s2, s3, g0, g1, g2, g3, dinv_ref, b_ref, w_ref) = refs[:11]
    outs = refs[11:]
    dinv = dinv_ref[...]
    sfull = jnp.concatenate([s0[...], s1[...], s2[...], s3[...]], axis=1)
    gfull = jnp.concatenate([g0[...], g1[...], g2[...], g3[...]], axis=1)
    h = dinv * (sfull + gfull) + b_ref[...]
    h = jnp.maximum(h, 0.0)
    t = jnp.dot(h, w_ref[...], preferred_element_type=jnp.float32)
    t = t * dinv
    if split_out:
        for p in range(NP):
            outs[p][...] = t[:, p * W:(p + 1) * W]
    else:
        outs[0][...] = t


def _tc_mid(splanes, gplanes, dinv, b, Wm, split_out):
    kout = Wm.shape[1]
    if split_out:
        out_specs = _plane_specs()
        out_shape = [jax.ShapeDtypeStruct((N_PAD, W), jnp.float32)
                     for _ in range(NP)]
    else:
        out_specs = [pl.BlockSpec((_TB, kout), lambda i: (i, 0))]
        out_shape = [jax.ShapeDtypeStruct((N_PAD, kout), jnp.float32)]
    return pl.pallas_call(
        functools.partial(_t2_body, split_out=split_out),
        grid=(_TGRID,),
        in_specs=_plane_specs() + _plane_specs() + [
            pl.BlockSpec((_TB, 1), lambda i: (i, 0)),
            pl.BlockSpec((1, D), lambda i: (0, 0)),
            pl.BlockSpec((D, kout), lambda i: (0, 0)),
        ],
        out_specs=out_specs,
        out_shape=out_shape,
    )(*splanes, *gplanes, dinv, b, Wm)


def _t4_body(p_ref, g_ref, dinv_ref, b_ref, out_ref):
    sarr = jnp.sum(p_ref[...], axis=1, keepdims=True)
    dinv = dinv_ref[...]
    h = dinv * (sarr + g_ref[...]) + b_ref[...]
    out_ref[...] = jnp.maximum(h, 0.0)


def _tc_final(p, g, dinv, b3):
    return pl.pallas_call(
        _t4_body,
        grid=(_TGRID,),
        in_specs=[
            pl.BlockSpec((_TB, NW), lambda i: (i, 0)),
            pl.BlockSpec((_TB, 1), lambda i: (i, 0)),
            pl.BlockSpec((_TB, 1), lambda i: (i, 0)),
            pl.BlockSpec((1, 1), lambda i: (0, 0)),
        ],
        out_specs=pl.BlockSpec((_TB, 1), lambda i: (i, 0)),
        out_shape=jax.ShapeDtypeStruct((N_PAD, 1), jnp.float32),
    )(p, g, dinv, b3)


@jax.jit
def kernel(x, edge_index, edge_attr, W1, b1, W2, b2, W3, b3):
    src = edge_index[0]
    dst = edge_index[1]
    ew = jnp.squeeze(edge_attr)

    pad = E_PAD - E
    src_p = jnp.concatenate([src, jnp.zeros((pad,), jnp.int32)])
    dst_p = jnp.concatenate([dst, jnp.zeros((pad,), jnp.int32)])
    ew_p = jnp.concatenate([ew, jnp.zeros((pad,), jnp.float32)])
    src2 = src_p.reshape(NW * NCHUNK, CHUNK)
    dst2 = dst_p.reshape(NW * NCHUNK, CHUNK)
    ew2 = ew_p.reshape(NW * NCHUNK, CHUNK)

    x_pad = jnp.pad(x, ((0, N_PAD - N), (0, 0)))

    psrc, pdst, pew, pcnt = _part_kernel(src2, dst2, ew2)

    degp = _deg_kernel(dst2, ew2).reshape(NW, N_PAD).T

    t1 = _tc_first(degp, x_pad, W1)
    dinv, g1p = t1[0], t1[1:]

    s1 = _prop_kernel(psrc, pdst, pew, pcnt, *g1p).reshape(NP, N_PAD, W)

    g2p = _tc_mid(list(s1), list(g1p), dinv, b1.reshape(1, D), W2, True)

    s2 = _prop_kernel(psrc, pdst, pew, pcnt, *g2p).reshape(NP, N_PAD, W)

    (g3,) = _tc_mid(list(s2), list(g2p), dinv, b2.reshape(1, D), W3, False)

    s3 = _prop1_kernel(src2, dst2, ew2, g3[:, 0]).reshape(NW, N_PAD).T

    out = _tc_final(s3, g3, dinv, b3.reshape(1, 1))
    return jnp.squeeze(out[:N])


# speedup vs baseline: 1.5834x; 1.1107x over previous
import functools

import jax
import jax.numpy as jnp
from jax import lax
from jax.experimental import pallas as pl
from jax.experimental.pallas import tpu as pltpu
from jax.experimental.pallas import tpu_sc as plsc

N = 10000
E = 320000
D = 128

NC = 2
NS = 16
NW = NC * NS
L = 16

CHUNK = 128
EPT = -(-E // (NW * CHUNK * 8)) * CHUNK * 8
E_PAD = EPT * NW
NCHUNK = EPT // CHUNK
N_PAD = 10240
QB = (0, 2048, 4096, 7168, 10240)
Q0SZ = 2048
Q1SZ = 3072
QMAX = 3072
FL = N_PAD // NS
PFL0 = Q0SZ // NS
PFL1 = Q1SZ // NS
BKT = EPT + 1024
BKTC = BKT // CHUNK
W = 32
NP = D // W

_mesh = plsc.VectorSubcoreMesh(core_axis_name="c", subcore_axis_name="s")
_sc_params = pltpu.CompilerParams(needs_layout_passes=False)


def _zero_1d(buf, n):
    def body(i, _):
        buf[pl.ds(i * L, L)] = jnp.zeros((L,), jnp.float32)
        return 0
    lax.fori_loop(0, n // L, body, 0)


@functools.partial(
    pl.kernel,
    out_type=[
        jax.ShapeDtypeStruct((NW * 4 * BKTC, CHUNK), jnp.int32),
        jax.ShapeDtypeStruct((NW * 4 * BKTC, CHUNK), jnp.int32),
        jax.ShapeDtypeStruct((NW * 4 * BKTC, CHUNK), jnp.int32),
        jax.ShapeDtypeStruct((NW * L,), jnp.int32),
    ],
    mesh=_mesh,
    compiler_params=_sc_params,
    scratch_types=[
        pltpu.VMEM((NCHUNK, CHUNK), jnp.int32),
        pltpu.VMEM((NCHUNK, CHUNK), jnp.int32),
        pltpu.VMEM((NCHUNK, CHUNK), jnp.float32),
        pltpu.VMEM((BKT,), jnp.int32),
        pltpu.VMEM((BKT,), jnp.int32),
        pltpu.VMEM((BKT,), jnp.int32),
        pltpu.VMEM((BKT,), jnp.int32),
        pltpu.VMEM((BKT,), jnp.float32),
        pltpu.VMEM((BKT,), jnp.float32),
        pltpu.VMEM((BKTC, CHUNK), jnp.int32),
        pltpu.VMEM((L,), jnp.int32),
    ],
)
def _part_kernel(src_hbm, dst_hbm, ew_hbm,
                 psrc_hbm, pdst_hbm, pew_hbm, pcnt_hbm,
                 srcs_v, dsts_v, ews_v,
                 bs0_v, bs1_v, bd0_v, bd1_v, bw0_v, bw1_v, d2_v, cnt_v):
    c = lax.axis_index("c")
    s = lax.axis_index("s")
    wid = c * NS + s

    pltpu.sync_copy(src_hbm.at[pl.ds(wid * NCHUNK, NCHUNK)], srcs_v)
    pltpu.sync_copy(dst_hbm.at[pl.ds(wid * NCHUNK, NCHUNK)], dsts_v)
    pltpu.sync_copy(ew_hbm.at[pl.ds(wid * NCHUNK, NCHUNK)], ews_v)

    zi = jnp.zeros((L,), jnp.int32)
    zf = jnp.zeros((L,), jnp.float32)

    def flush2d(flat, out2d, q, is_f32):
        def redist(r, _):
            for j in range(CHUNK // L):
                v = flat[pl.ds(r * CHUNK + j * L, L)]
                if is_f32:
                    v = plsc.bitcast(v, jnp.int32)
                d2_v[r, pl.ds(j * L, L)] = v
            return 0
        lax.fori_loop(0, BKTC, redist, 0)
        pltpu.sync_copy(d2_v, out2d.at[pl.ds((wid * 4 + q) * BKTC, BKTC)])

    qcnt = [None] * 4
    for r in (0, 1):
        lo0, hi0, hi1 = QB[2 * r], QB[2 * r + 1], QB[2 * r + 2]

        def group_body(k, offs):
            off0, off1 = offs
            i = k // (CHUNK // L)
            g = k % (CHUNK // L)
            sl = pl.ds(g * L, L)
            s16 = srcs_v[i, sl]
            d16 = dsts_v[i, sl]
            w16 = ews_v[i, sl]
            m0 = jnp.logical_and(d16 >= lo0, d16 < hi0)
            m1 = jnp.logical_and(d16 >= hi0, d16 < hi1)
            plsc.store_compressed(bs0_v.at[pl.ds(off0, L)], s16, mask=m0)
            plsc.store_compressed(bd0_v.at[pl.ds(off0, L)], d16 - lo0, mask=m0)
            plsc.store_compressed(bw0_v.at[pl.ds(off0, L)], w16, mask=m0)
            plsc.store_compressed(bs1_v.at[pl.ds(off1, L)], s16, mask=m1)
            plsc.store_compressed(bd1_v.at[pl.ds(off1, L)], d16 - hi0, mask=m1)
            plsc.store_compressed(bw1_v.at[pl.ds(off1, L)], w16, mask=m1)
            n0 = jnp.max(plsc.all_reduce_population_count(m0))
            n1 = jnp.max(plsc.all_reduce_population_count(m1))
            return (off0 + n0, off1 + n1)

        off0, off1 = lax.fori_loop(0, EPT // L, group_body, (0, 0))

        def pad_body(k, _):
            bs0_v[pl.ds(off0 + k * L, L)] = zi
            bd0_v[pl.ds(off0 + k * L, L)] = zi
            bw0_v[pl.ds(off0 + k * L, L)] = zf
            bs1_v[pl.ds(off1 + k * L, L)] = zi
            bd1_v[pl.ds(off1 + k * L, L)] = zi
            bw1_v[pl.ds(off1 + k * L, L)] = zf
            return 0
        lax.fori_loop(0, CHUNK // L, pad_body, 0)

        flush2d(bs0_v, psrc_hbm, 2 * r, False)
        flush2d(bs1_v, psrc_hbm, 2 * r + 1, False)
        flush2d(bd0_v, pdst_hbm, 2 * r, False)
        flush2d(bd1_v, pdst_hbm, 2 * r + 1, False)
        flush2d(bw0_v, pew_hbm, 2 * r, True)
        flush2d(bw1_v, pew_hbm, 2 * r + 1, True)
        qcnt[2 * r] = off0
        qcnt[2 * r + 1] = off1

    lane = lax.iota(jnp.int32, L)
    cv = jnp.zeros((L,), jnp.int32)
    for q in range(4):
        cv = jnp.where(lane == q, qcnt[q], cv)
    cnt_v[pl.ds(0, L)] = cv
    pltpu.sync_copy(cnt_v, pcnt_hbm.at[pl.ds(wid * L, L)])


@functools.partial(
    pl.kernel,
    out_type=jax.ShapeDtypeStruct((NW * N_PAD,), jnp.float32),
    mesh=_mesh,
    compiler_params=_sc_params,
    scratch_types=[
        pltpu.VMEM((NCHUNK, CHUNK), jnp.int32),
        pltpu.VMEM((NCHUNK, CHUNK), jnp.float32),
        pltpu.VMEM((N_PAD,), jnp.float32),
    ],
)
def _deg_kernel(dst_hbm, ew_hbm, out_hbm, dsts_v, ews_v, acc_v):
    c = lax.axis_index("c")
    s = lax.axis_index("s")
    wid = c * NS + s

    _zero_1d(acc_v, N_PAD)
    pltpu.sync_copy(dst_hbm.at[pl.ds(wid * NCHUNK, NCHUNK)], dsts_v)
    pltpu.sync_copy(ew_hbm.at[pl.ds(wid * NCHUNK, NCHUNK)], ews_v)

    def chunk_body(i, _):
        for g in range(CHUNK // L):
            sl = pl.ds(g * L, L)
            idx = dsts_v[i, sl]
            w = ews_v[i, sl]
            plsc.addupdate_scatter(acc_v, [idx], w)
        return 0

    lax.fori_loop(0, NCHUNK, chunk_body, 0)
    pltpu.sync_copy(acc_v, out_hbm.at[pl.ds(wid * N_PAD, N_PAD)])


@functools.partial(
    pl.kernel,
    out_type=jax.ShapeDtypeStruct((NP * N_PAD, W), jnp.float32),
    mesh=_mesh,
    compiler_params=pltpu.CompilerParams(
        needs_layout_passes=False, use_tc_tiling_on_sc=False),
    scratch_types=[
        pltpu.VMEM((BKTC, CHUNK), jnp.int32),
        pltpu.VMEM((BKTC, CHUNK), jnp.int32),
        pltpu.VMEM((BKTC, CHUNK), jnp.int32),
        pltpu.VMEM((BKTC, CHUNK), jnp.int32),
        pltpu.VMEM((BKTC, CHUNK), jnp.int32),
        pltpu.VMEM((BKTC, CHUNK), jnp.int32),
        pltpu.VMEM((CHUNK, W), jnp.float32),
        pltpu.VMEM((CHUNK, W), jnp.float32),
        pltpu.VMEM((PFL1, W), jnp.float32),
        pltpu.VMEM((L,), jnp.int32),
        pltpu.VMEM_SHARED((N_PAD, W), jnp.float32),
        pltpu.VMEM_SHARED((QMAX, W), jnp.float32),
        pltpu.SemaphoreType.DMA,
        pltpu.SemaphoreType.DMA,
        pltpu.SemaphoreType.DMA,
        pltpu.SemaphoreType.DMA,
    ],
)
def _prop_kernel(psrc_hbm, pdst_hbm, pew_hbm, pcnt_hbm,
                 g0_hbm, g1_hbm, g2_hbm, g3_hbm, out_hbm,
                 bsa_v, bda_v, bwa_v, bsb_v, bdb_v, bwb_v,
                 rows0_v, rows1_v, stage_v, cnt_v, tab_sh, acc_sh,
                 gsem0, gsem1, ssem0, ssem1):
    c = lax.axis_index("c")
    s = lax.axis_index("s")

    lane = lax.iota(jnp.int32, L)
    rows = (rows0_v, rows1_v)
    gsem = (gsem0, gsem1)
    ssem = (ssem0, ssem1)
    gp = (g0_hbm, g1_hbm, g2_hbm, g3_hbm)
    TSL = N_PAD // NS

    def bucket_cnt(t, q):
        pltpu.sync_copy(pcnt_hbm.at[pl.ds(t * L, L)], cnt_v)
        return jnp.max(jnp.where(lane == q, cnt_v[pl.ds(0, L)], 0))

    def load_bucket(t, q, bs, bd, bw):
        base = (t * 4 + q) * BKTC
        pltpu.sync_copy(psrc_hbm.at[pl.ds(base, BKTC)], bs)
        pltpu.sync_copy(pew_hbm.at[pl.ds(base, BKTC)], bw)
        pltpu.sync_copy(pdst_hbm.at[pl.ds(base, BKTC)], bd)

    def run_bucket(nch, bs, bd, bw):
        def gather_start(i, b):
            pltpu.async_copy(tab_sh.at[bs.at[i]], rows[b], gsem[b])

        def gather_wait(b):
            pltpu.make_async_copy(tab_sh.at[bs.at[0]], rows[b], gsem[b]).wait()

        def scatter_start(i, b):
            pltpu.async_copy(rows[b], acc_sh.at[bd.at[i]], ssem[b], add=True)

        def scatter_wait(b):
            pltpu.make_async_copy(rows[b], acc_sh.at[bd.at[0]], ssem[b]).wait()

        pl.when(nch > 0)(lambda: gather_start(0, 0))

        def pair_body(i2, _):
            for b in (0, 1):
                i = 2 * i2 + b

                @pl.when(i < nch)
                def _process():
                    gather_wait(b)
                    pl.when(i >= 1)(lambda: scatter_wait(1 - b))
                    pl.when(i + 1 < nch)(lambda: gather_start(i + 1, 1 - b))

                    rv = rows[b]

                    def scale_body(e, _):
                        bc = plsc.bitcast(
                            plsc.load_gather(
                                bw,
                                [jnp.full((L,), i, jnp.int32),
                                 jnp.full((L,), e, jnp.int32)]),
                            jnp.float32)
                        for j in range(W // L):
                            sl = pl.ds(j * L, L)
                            rv[e, sl] = rv[e, sl] * bc
                        return 0
                    lax.fori_loop(0, CHUNK, scale_body, 0)

                    scatter_start(i, b)
            return 0

        lax.fori_loop(0, (nch + 1) // 2, pair_body, 0)
        pl.when(nch % 2 == 1)(lambda: scatter_wait(0))
        pl.when(jnp.logical_and(nch > 0, nch % 2 == 0))(
            lambda: scatter_wait(1))

    def zero_flush(p, p_outer, pflc, qbase, do_flush):
        if do_flush:
            pltpu.sync_copy(acc_sh.at[pl.ds(s * pflc, pflc)],
                            stage_v.at[pl.ds(0, pflc)])
            pltpu.sync_copy(
                stage_v.at[pl.ds(0, pflc)],
                out_hbm.at[pl.ds(p * N_PAD + qbase + s * pflc, pflc)])
        else:
            def zero_body(i, _):
                for j in range(W // L):
                    stage_v[i, pl.ds(j * L, L)] = jnp.zeros((L,), jnp.float32)
                return 0
            lax.fori_loop(0, pflc, zero_body, 0)
            pltpu.sync_copy(stage_v.at[pl.ds(0, pflc)],
                            acc_sh.at[pl.ds(s * pflc, pflc)])

    for p_outer in (0, 1):
        q = 2 * c + p_outer
        ncha = lax.div(bucket_cnt(2 * s, q) + (CHUNK - 1), CHUNK)
        nchb = lax.div(bucket_cnt(2 * s + 1, q) + (CHUNK - 1), CHUNK)
        load_bucket(2 * s, q, bsa_v, bda_v, bwa_v)
        load_bucket(2 * s + 1, q, bsb_v, bdb_v, bwb_v)

        for p in range(NP):
            pltpu.sync_copy(gp[p].at[pl.ds(s * TSL, TSL)],
                            tab_sh.at[pl.ds(s * TSL, TSL)])

            pl.when(c == 0)(
                lambda: zero_flush(p, p_outer, PFL0, QB[p_outer], False))
            pl.when(c == 1)(
                lambda: zero_flush(p, p_outer, PFL1, QB[2 + p_outer], False))
            plsc.subcore_barrier()

            run_bucket(ncha, bsa_v, bda_v, bwa_v)
            run_bucket(nchb, bsb_v, bdb_v, bwb_v)
            plsc.subcore_barrier()

            pl.when(c == 0)(
                lambda: zero_flush(p, p_outer, PFL0, QB[p_outer], True))
            pl.when(c == 1)(
                lambda: zero_flush(p, p_outer, PFL1, QB[2 + p_outer], True))


@functools.partial(
    pl.kernel,
    out_type=jax.ShapeDtypeStruct((NW * N_PAD,), jnp.float32),
    mesh=_mesh,
    compiler_params=_sc_params,
    scratch_types=[
        pltpu.VMEM((NCHUNK, CHUNK), jnp.int32),
        pltpu.VMEM((NCHUNK, CHUNK), jnp.int32),
        pltpu.VMEM((NCHUNK, CHUNK), jnp.float32),
        pltpu.VMEM((N_PAD,), jnp.float32),
        pltpu.VMEM((N_PAD,), jnp.float32),
    ],
)
def _prop1_kernel(src_hbm, dst_hbm, ew_hbm, g_hbm, out_hbm,
                  srcs_v, dsts_v, ews_v, tab_v, acc_v):
    c = lax.axis_index("c")
    s = lax.axis_index("s")
    wid = c * NS + s

    _zero_1d(acc_v, N_PAD)
    pltpu.sync_copy(g_hbm, tab_v)
    pltpu.sync_copy(src_hbm.at[pl.ds(wid * NCHUNK, NCHUNK)], srcs_v)
    pltpu.sync_copy(dst_hbm.at[pl.ds(wid * NCHUNK, NCHUNK)], dsts_v)
    pltpu.sync_copy(ew_hbm.at[pl.ds(wid * NCHUNK, NCHUNK)], ews_v)

    def chunk_body(i, _):
        for g in range(CHUNK // L):
            sl = pl.ds(g * L, L)
            sidx = srcs_v[i, sl]
            didx = dsts_v[i, sl]
            w = ews_v[i, sl]
            vals = plsc.load_gather(tab_v, [sidx])
            plsc.addupdate_scatter(acc_v, [didx], vals * w)
        return 0

    lax.fori_loop(0, NCHUNK, chunk_body, 0)
    pltpu.sync_copy(acc_v, out_hbm.at[pl.ds(wid * N_PAD, N_PAD)])


_TB = 1024
_TGRID = N_PAD // _TB


def _plane_specs():
    return [pl.BlockSpec((_TB, W), lambda i: (i, 0)) for _ in range(NP)]


def _t1_body(p_ref, x_ref, w_ref, dinv_ref, *g_refs):
    deg = 1.0 + jnp.sum(p_ref[...], axis=1, keepdims=True)
    dinv = jnp.where(deg > 0, lax.rsqrt(deg), 0.0)
    dinv_ref[...] = dinv
    t = jnp.dot(x_ref[...], w_ref[...], preferred_element_type=jnp.float32)
    t = t * dinv
    for p in range(NP):
        g_refs[p][...] = t[:, p * W:(p + 1) * W]


def _tc_first(pdeg, x, W1):
    return pl.pallas_call(
        _t1_body,
        grid=(_TGRID,),
        in_specs=[
            pl.BlockSpec((_TB, NW), lambda i: (i, 0)),
            pl.BlockSpec((_TB, D), lambda i: (i, 0)),
            pl.BlockSpec((D, D), lambda i: (0, 0)),
        ],
        out_specs=[pl.BlockSpec((_TB, 1), lambda i: (i, 0))] + _plane_specs(),
        out_shape=[jax.ShapeDtypeStruct((N_PAD, 1), jnp.float32)]
        + [jax.ShapeDtypeStruct((N_PAD, W), jnp.float32) for _ in range(NP)],
    )(pdeg, x, W1)


def _t2_body(*refs, split_out):
    (s0, s1, s2, s3, g0, g1, g2, g3, dinv_ref, b_ref, w_ref) = refs[:11]
    outs = refs[11:]
    dinv = dinv_ref[...]
    sfull = jnp.concatenate([s0[...], s1[...], s2[...], s3[...]], axis=1)
    gfull = jnp.concatenate([g0[...], g1[...], g2[...], g3[...]], axis=1)
    h = dinv * (sfull + gfull) + b_ref[...]
    h = jnp.maximum(h, 0.0)
    t = jnp.dot(h, w_ref[...], preferred_element_type=jnp.float32)
    t = t * dinv
    if split_out:
        for p in range(NP):
            outs[p][...] = t[:, p * W:(p + 1) * W]
    else:
        outs[0][...] = t


def _tc_mid(splanes, gplanes, dinv, b, Wm, split_out):
    kout = Wm.shape[1]
    if split_out:
        out_specs = _plane_specs()
        out_shape = [jax.ShapeDtypeStruct((N_PAD, W), jnp.float32)
                     for _ in range(NP)]
    else:
        out_specs = [pl.BlockSpec((_TB, kout), lambda i: (i, 0))]
        out_shape = [jax.ShapeDtypeStruct((N_PAD, kout), jnp.float32)]
    return pl.pallas_call(
        functools.partial(_t2_body, split_out=split_out),
        grid=(_TGRID,),
        in_specs=_plane_specs() + _plane_specs() + [
            pl.BlockSpec((_TB, 1), lambda i: (i, 0)),
            pl.BlockSpec((1, D), lambda i: (0, 0)),
            pl.BlockSpec((D, kout), lambda i: (0, 0)),
        ],
        out_specs=out_specs,
        out_shape=out_shape,
    )(*splanes, *gplanes, dinv, b, Wm)


def _t4_body(p_ref, g_ref, dinv_ref, b_ref, out_ref):
    sarr = jnp.sum(p_ref[...], axis=1, keepdims=True)
    dinv = dinv_ref[...]
    h = dinv * (sarr + g_ref[...]) + b_ref[...]
    out_ref[...] = jnp.maximum(h, 0.0)


def _tc_final(p, g, dinv, b3):
    return pl.pallas_call(
        _t4_body,
        grid=(_TGRID,),
        in_specs=[
            pl.BlockSpec((_TB, NW), lambda i: (i, 0)),
            pl.BlockSpec((_TB, 1), lambda i: (i, 0)),
            pl.BlockSpec((_TB, 1), lambda i: (i, 0)),
            pl.BlockSpec((1, 1), lambda i: (0, 0)),
        ],
        out_specs=pl.BlockSpec((_TB, 1), lambda i: (i, 0)),
        out_shape=jax.ShapeDtypeStruct((N_PAD, 1), jnp.float32),
    )(p, g, dinv, b3)


@jax.jit
def kernel(x, edge_index, edge_attr, W1, b1, W2, b2, W3, b3):
    src = edge_index[0]
    dst = edge_index[1]
    ew = jnp.squeeze(edge_attr)

    pad = E_PAD - E
    src_p = jnp.concatenate([src, jnp.zeros((pad,), jnp.int32)])
    dst_p = jnp.concatenate([dst, jnp.zeros((pad,), jnp.int32)])
    ew_p = jnp.concatenate([ew, jnp.zeros((pad,), jnp.float32)])
    src2 = src_p.reshape(NW * NCHUNK, CHUNK)
    dst2 = dst_p.reshape(NW * NCHUNK, CHUNK)
    ew2 = ew_p.reshape(NW * NCHUNK, CHUNK)

    x_pad = jnp.pad(x, ((0, N_PAD - N), (0, 0)))

    psrc, pdst, pew, pcnt = _part_kernel(src2, dst2, ew2)

    degp = _deg_kernel(dst2, ew2).reshape(NW, N_PAD).T

    t1 = _tc_first(degp, x_pad, W1)
    dinv, g1p = t1[0], t1[1:]

    s1 = _prop_kernel(psrc, pdst, pew, pcnt, *g1p).reshape(NP, N_PAD, W)

    g2p = _tc_mid(list(s1), list(g1p), dinv, b1.reshape(1, D), W2, True)

    s2 = _prop_kernel(psrc, pdst, pew, pcnt, *g2p).reshape(NP, N_PAD, W)

    (g3,) = _tc_mid(list(s2), list(g2p), dinv, b2.reshape(1, D), W3, False)

    s3 = _prop1_kernel(src2, dst2, ew2, g3[:, 0]).reshape(NW, N_PAD).T

    out = _tc_final(s3, g3, dinv, b3.reshape(1, 1))
    return jnp.squeeze(out[:N])


# speedup vs baseline: 1.6128x; 1.0186x over previous
import functools

import jax
import jax.numpy as jnp
from jax import lax
from jax.experimental import pallas as pl
from jax.experimental.pallas import tpu as pltpu
from jax.experimental.pallas import tpu_sc as plsc

N = 10000
E = 320000
D = 128

NC = 2
NS = 16
NW = NC * NS
L = 16

CHUNK = 128
EPT = -(-E // (NW * CHUNK * 8)) * CHUNK * 8
E_PAD = EPT * NW
NCHUNK = EPT // CHUNK
N_PAD = 10240
QB = (0, 1920, 3840, 7040, 10240)
Q0SZ = 1920
Q1SZ = 3200
QMAX = 3200
FL = N_PAD // NS
PFL0 = Q0SZ // NS
PFL1 = Q1SZ // NS
BKT = EPT + 1024
BKTC = BKT // CHUNK
W = 32
NP = D // W

_mesh = plsc.VectorSubcoreMesh(core_axis_name="c", subcore_axis_name="s")
_sc_params = pltpu.CompilerParams(needs_layout_passes=False)


def _zero_1d(buf, n):
    def body(i, _):
        buf[pl.ds(i * L, L)] = jnp.zeros((L,), jnp.float32)
        return 0
    lax.fori_loop(0, n // L, body, 0)


@functools.partial(
    pl.kernel,
    out_type=[
        jax.ShapeDtypeStruct((NW * 4 * BKTC, CHUNK), jnp.int32),
        jax.ShapeDtypeStruct((NW * 4 * BKTC, CHUNK), jnp.int32),
        jax.ShapeDtypeStruct((NW * 4 * BKTC, CHUNK), jnp.int32),
        jax.ShapeDtypeStruct((NW * L,), jnp.int32),
    ],
    mesh=_mesh,
    compiler_params=_sc_params,
    scratch_types=[
        pltpu.VMEM((NCHUNK, CHUNK), jnp.int32),
        pltpu.VMEM((NCHUNK, CHUNK), jnp.int32),
        pltpu.VMEM((NCHUNK, CHUNK), jnp.float32),
        pltpu.VMEM((BKT,), jnp.int32),
        pltpu.VMEM((BKT,), jnp.int32),
        pltpu.VMEM((BKT,), jnp.int32),
        pltpu.VMEM((BKT,), jnp.int32),
        pltpu.VMEM((BKT,), jnp.float32),
        pltpu.VMEM((BKT,), jnp.float32),
        pltpu.VMEM((BKTC, CHUNK), jnp.int32),
        pltpu.VMEM((L,), jnp.int32),
    ],
)
def _part_kernel(src_hbm, dst_hbm, ew_hbm,
                 psrc_hbm, pdst_hbm, pew_hbm, pcnt_hbm,
                 srcs_v, dsts_v, ews_v,
                 bs0_v, bs1_v, bd0_v, bd1_v, bw0_v, bw1_v, d2_v, cnt_v):
    c = lax.axis_index("c")
    s = lax.axis_index("s")
    wid = c * NS + s

    pltpu.sync_copy(src_hbm.at[pl.ds(wid * NCHUNK, NCHUNK)], srcs_v)
    pltpu.sync_copy(dst_hbm.at[pl.ds(wid * NCHUNK, NCHUNK)], dsts_v)
    pltpu.sync_copy(ew_hbm.at[pl.ds(wid * NCHUNK, NCHUNK)], ews_v)

    zi = jnp.zeros((L,), jnp.int32)
    zf = jnp.zeros((L,), jnp.float32)

    def flush2d(flat, out2d, q, is_f32):
        def redist(r, _):
            for j in range(CHUNK // L):
                v = flat[pl.ds(r * CHUNK + j * L, L)]
                if is_f32:
                    v = plsc.bitcast(v, jnp.int32)
                d2_v[r, pl.ds(j * L, L)] = v
            return 0
        lax.fori_loop(0, BKTC, redist, 0)
        pltpu.sync_copy(d2_v, out2d.at[pl.ds((wid * 4 + q) * BKTC, BKTC)])

    qcnt = [None] * 4
    for r in (0, 1):
        lo0, hi0, hi1 = QB[2 * r], QB[2 * r + 1], QB[2 * r + 2]

        def group_body(k, offs):
            off0, off1 = offs
            i = k // (CHUNK // L)
            g = k % (CHUNK // L)
            sl = pl.ds(g * L, L)
            s16 = srcs_v[i, sl]
            d16 = dsts_v[i, sl]
            w16 = ews_v[i, sl]
            m0 = jnp.logical_and(d16 >= lo0, d16 < hi0)
            m1 = jnp.logical_and(d16 >= hi0, d16 < hi1)
            plsc.store_compressed(bs0_v.at[pl.ds(off0, L)], s16, mask=m0)
            plsc.store_compressed(bd0_v.at[pl.ds(off0, L)], d16 - lo0, mask=m0)
            plsc.store_compressed(bw0_v.at[pl.ds(off0, L)], w16, mask=m0)
            plsc.store_compressed(bs1_v.at[pl.ds(off1, L)], s16, mask=m1)
            plsc.store_compressed(bd1_v.at[pl.ds(off1, L)], d16 - hi0, mask=m1)
            plsc.store_compressed(bw1_v.at[pl.ds(off1, L)], w16, mask=m1)
            n0 = jnp.max(plsc.all_reduce_population_count(m0))
            n1 = jnp.max(plsc.all_reduce_population_count(m1))
            return (off0 + n0, off1 + n1)

        off0, off1 = lax.fori_loop(0, EPT // L, group_body, (0, 0))

        def pad_body(k, _):
            bs0_v[pl.ds(off0 + k * L, L)] = zi
            bd0_v[pl.ds(off0 + k * L, L)] = zi
            bw0_v[pl.ds(off0 + k * L, L)] = zf
            bs1_v[pl.ds(off1 + k * L, L)] = zi
            bd1_v[pl.ds(off1 + k * L, L)] = zi
            bw1_v[pl.ds(off1 + k * L, L)] = zf
            return 0
        lax.fori_loop(0, CHUNK // L, pad_body, 0)

        flush2d(bs0_v, psrc_hbm, 2 * r, False)
        flush2d(bs1_v, psrc_hbm, 2 * r + 1, False)
        flush2d(bd0_v, pdst_hbm, 2 * r, False)
        flush2d(bd1_v, pdst_hbm, 2 * r + 1, False)
        flush2d(bw0_v, pew_hbm, 2 * r, True)
        flush2d(bw1_v, pew_hbm, 2 * r + 1, True)
        qcnt[2 * r] = off0
        qcnt[2 * r + 1] = off1

    lane = lax.iota(jnp.int32, L)
    cv = jnp.zeros((L,), jnp.int32)
    for q in range(4):
        cv = jnp.where(lane == q, qcnt[q], cv)
    cnt_v[pl.ds(0, L)] = cv
    pltpu.sync_copy(cnt_v, pcnt_hbm.at[pl.ds(wid * L, L)])


@functools.partial(
    pl.kernel,
    out_type=jax.ShapeDtypeStruct((NW * N_PAD,), jnp.float32),
    mesh=_mesh,
    compiler_params=_sc_params,
    scratch_types=[
        pltpu.VMEM((NCHUNK, CHUNK), jnp.int32),
        pltpu.VMEM((NCHUNK, CHUNK), jnp.float32),
        pltpu.VMEM((N_PAD,), jnp.float32),
    ],
)
def _deg_kernel(dst_hbm, ew_hbm, out_hbm, dsts_v, ews_v, acc_v):
    c = lax.axis_index("c")
    s = lax.axis_index("s")
    wid = c * NS + s

    _zero_1d(acc_v, N_PAD)
    pltpu.sync_copy(dst_hbm.at[pl.ds(wid * NCHUNK, NCHUNK)], dsts_v)
    pltpu.sync_copy(ew_hbm.at[pl.ds(wid * NCHUNK, NCHUNK)], ews_v)

    def chunk_body(i, _):
        for g in range(CHUNK // L):
            sl = pl.ds(g * L, L)
            idx = dsts_v[i, sl]
            w = ews_v[i, sl]
            plsc.addupdate_scatter(acc_v, [idx], w)
        return 0

    lax.fori_loop(0, NCHUNK, chunk_body, 0)
    pltpu.sync_copy(acc_v, out_hbm.at[pl.ds(wid * N_PAD, N_PAD)])


@functools.partial(
    pl.kernel,
    out_type=jax.ShapeDtypeStruct((NP * N_PAD, W), jnp.float32),
    mesh=_mesh,
    compiler_params=pltpu.CompilerParams(
        needs_layout_passes=False, use_tc_tiling_on_sc=False),
    scratch_types=[
        pltpu.VMEM((BKTC, CHUNK), jnp.int32),
        pltpu.VMEM((BKTC, CHUNK), jnp.int32),
        pltpu.VMEM((BKTC, CHUNK), jnp.int32),
        pltpu.VMEM((BKTC, CHUNK), jnp.int32),
        pltpu.VMEM((BKTC, CHUNK), jnp.int32),
        pltpu.VMEM((BKTC, CHUNK), jnp.int32),
        pltpu.VMEM((CHUNK, W), jnp.float32),
        pltpu.VMEM((CHUNK, W), jnp.float32),
        pltpu.VMEM((PFL1, W), jnp.float32),
        pltpu.VMEM((L,), jnp.int32),
        pltpu.VMEM_SHARED((N_PAD, W), jnp.float32),
        pltpu.VMEM_SHARED((QMAX, W), jnp.float32),
        pltpu.SemaphoreType.DMA,
        pltpu.SemaphoreType.DMA,
        pltpu.SemaphoreType.DMA,
        pltpu.SemaphoreType.DMA,
    ],
)
def _prop_kernel(psrc_hbm, pdst_hbm, pew_hbm, pcnt_hbm,
                 g0_hbm, g1_hbm, g2_hbm, g3_hbm, out_hbm,
                 bsa_v, bda_v, bwa_v, bsb_v, bdb_v, bwb_v,
                 rows0_v, rows1_v, stage_v, cnt_v, tab_sh, acc_sh,
                 gsem0, gsem1, ssem0, ssem1):
    c = lax.axis_index("c")
    s = lax.axis_index("s")

    lane = lax.iota(jnp.int32, L)
    rows = (rows0_v, rows1_v)
    gsem = (gsem0, gsem1)
    ssem = (ssem0, ssem1)
    gp = (g0_hbm, g1_hbm, g2_hbm, g3_hbm)
    TSL = N_PAD // NS

    def bucket_cnt(t, q):
        pltpu.sync_copy(pcnt_hbm.at[pl.ds(t * L, L)], cnt_v)
        return jnp.max(jnp.where(lane == q, cnt_v[pl.ds(0, L)], 0))

    def load_bucket(t, q, bs, bd, bw):
        base = (t * 4 + q) * BKTC
        pltpu.sync_copy(psrc_hbm.at[pl.ds(base, BKTC)], bs)
        pltpu.sync_copy(pew_hbm.at[pl.ds(base, BKTC)], bw)
        pltpu.sync_copy(pdst_hbm.at[pl.ds(base, BKTC)], bd)

    def run_bucket(nch, bs, bd, bw):
        def gather_start(i, b):
            pltpu.async_copy(tab_sh.at[bs.at[i]], rows[b], gsem[b])

        def gather_wait(b):
            pltpu.make_async_copy(tab_sh.at[bs.at[0]], rows[b], gsem[b]).wait()

        def scatter_start(i, b):
            pltpu.async_copy(rows[b], acc_sh.at[bd.at[i]], ssem[b], add=True)

        def scatter_wait(b):
            pltpu.make_async_copy(rows[b], acc_sh.at[bd.at[0]], ssem[b]).wait()

        pl.when(nch > 0)(lambda: gather_start(0, 0))

        def pair_body(i2, _):
            for b in (0, 1):
                i = 2 * i2 + b

                @pl.when(i < nch)
                def _process():
                    gather_wait(b)
                    pl.when(i >= 1)(lambda: scatter_wait(1 - b))
                    pl.when(i + 1 < nch)(lambda: gather_start(i + 1, 1 - b))

                    rv = rows[b]

                    def scale_body(e, _):
                        bc = plsc.bitcast(
                            plsc.load_gather(
                                bw,
                                [jnp.full((L,), i, jnp.int32),
                                 jnp.full((L,), e, jnp.int32)]),
                            jnp.float32)
                        for j in range(W // L):
                            sl = pl.ds(j * L, L)
                            rv[e, sl] = rv[e, sl] * bc
                        return 0
                    lax.fori_loop(0, CHUNK, scale_body, 0)

                    scatter_start(i, b)
            return 0

        lax.fori_loop(0, (nch + 1) // 2, pair_body, 0)
        pl.when(nch % 2 == 1)(lambda: scatter_wait(0))
        pl.when(jnp.logical_and(nch > 0, nch % 2 == 0))(
            lambda: scatter_wait(1))

    def zero_flush(p, p_outer, pflc, qbase, do_flush):
        if do_flush:
            pltpu.sync_copy(acc_sh.at[pl.ds(s * pflc, pflc)],
                            stage_v.at[pl.ds(0, pflc)])
            pltpu.sync_copy(
                stage_v.at[pl.ds(0, pflc)],
                out_hbm.at[pl.ds(p * N_PAD + qbase + s * pflc, pflc)])
        else:
            def zero_body(i, _):
                for j in range(W // L):
                    stage_v[i, pl.ds(j * L, L)] = jnp.zeros((L,), jnp.float32)
                return 0
            lax.fori_loop(0, pflc, zero_body, 0)
            pltpu.sync_copy(stage_v.at[pl.ds(0, pflc)],
                            acc_sh.at[pl.ds(s * pflc, pflc)])

    for p_outer in (0, 1):
        q = 2 * c + p_outer
        ncha = lax.div(bucket_cnt(2 * s, q) + (CHUNK - 1), CHUNK)
        nchb = lax.div(bucket_cnt(2 * s + 1, q) + (CHUNK - 1), CHUNK)
        load_bucket(2 * s, q, bsa_v, bda_v, bwa_v)
        load_bucket(2 * s + 1, q, bsb_v, bdb_v, bwb_v)

        for p in range(NP):
            pltpu.sync_copy(gp[p].at[pl.ds(s * TSL, TSL)],
                            tab_sh.at[pl.ds(s * TSL, TSL)])

            pl.when(c == 0)(
                lambda: zero_flush(p, p_outer, PFL0, QB[p_outer], False))
            pl.when(c == 1)(
                lambda: zero_flush(p, p_outer, PFL1, QB[2 + p_outer], False))
            plsc.subcore_barrier()

            run_bucket(ncha, bsa_v, bda_v, bwa_v)
            run_bucket(nchb, bsb_v, bdb_v, bwb_v)
            plsc.subcore_barrier()

            pl.when(c == 0)(
                lambda: zero_flush(p, p_outer, PFL0, QB[p_outer], True))
            pl.when(c == 1)(
                lambda: zero_flush(p, p_outer, PFL1, QB[2 + p_outer], True))


@functools.partial(
    pl.kernel,
    out_type=jax.ShapeDtypeStruct((NW * N_PAD,), jnp.float32),
    mesh=_mesh,
    compiler_params=_sc_params,
    scratch_types=[
        pltpu.VMEM((NCHUNK, CHUNK), jnp.int32),
        pltpu.VMEM((NCHUNK, CHUNK), jnp.int32),
        pltpu.VMEM((NCHUNK, CHUNK), jnp.float32),
        pltpu.VMEM((N_PAD,), jnp.float32),
        pltpu.VMEM((N_PAD,), jnp.float32),
    ],
)
def _prop1_kernel(src_hbm, dst_hbm, ew_hbm, g_hbm, out_hbm,
                  srcs_v, dsts_v, ews_v, tab_v, acc_v):
    c = lax.axis_index("c")
    s = lax.axis_index("s")
    wid = c * NS + s

    _zero_1d(acc_v, N_PAD)
    pltpu.sync_copy(g_hbm, tab_v)
    pltpu.sync_copy(src_hbm.at[pl.ds(wid * NCHUNK, NCHUNK)], srcs_v)
    pltpu.sync_copy(dst_hbm.at[pl.ds(wid * NCHUNK, NCHUNK)], dsts_v)
    pltpu.sync_copy(ew_hbm.at[pl.ds(wid * NCHUNK, NCHUNK)], ews_v)

    def chunk_body(i, _):
        for g in range(CHUNK // L):
            sl = pl.ds(g * L, L)
            sidx = srcs_v[i, sl]
            didx = dsts_v[i, sl]
            w = ews_v[i, sl]
            vals = plsc.load_gather(tab_v, [sidx])
            plsc.addupdate_scatter(acc_v, [didx], vals * w)
        return 0

    lax.fori_loop(0, NCHUNK, chunk_body, 0)
    pltpu.sync_copy(acc_v, out_hbm.at[pl.ds(wid * N_PAD, N_PAD)])


_TB = 1024
_TGRID = N_PAD // _TB


def _plane_specs():
    return [pl.BlockSpec((_TB, W), lambda i: (i, 0)) for _ in range(NP)]


def _t1_body(p_ref, x_ref, w_ref, dinv_ref, *g_refs):
    deg = 1.0 + jnp.sum(p_ref[...], axis=1, keepdims=True)
    dinv = jnp.where(deg > 0, lax.rsqrt(deg), 0.0)
    dinv_ref[...] = dinv
    t = jnp.dot(x_ref[...], w_ref[...], preferred_element_type=jnp.float32)
    t = t * dinv
    for p in range(NP):
        g_refs[p][...] = t[:, p * W:(p + 1) * W]


def _tc_first(pdeg, x, W1):
    return pl.pallas_call(
        _t1_body,
        grid=(_TGRID,),
        in_specs=[
            pl.BlockSpec((_TB, NW), lambda i: (i, 0)),
            pl.BlockSpec((_TB, D), lambda i: (i, 0)),
            pl.BlockSpec((D, D), lambda i: (0, 0)),
        ],
        out_specs=[pl.BlockSpec((_TB, 1), lambda i: (i, 0))] + _plane_specs(),
        out_shape=[jax.ShapeDtypeStruct((N_PAD, 1), jnp.float32)]
        + [jax.ShapeDtypeStruct((N_PAD, W), jnp.float32) for _ in range(NP)],
    )(pdeg, x, W1)


def _t2_body(*refs, split_out):
    (s0, s1, s2, s3, g0, g1, g2, g3, dinv_ref, b_ref, w_ref) = refs[:11]
    outs = refs[11:]
    dinv = dinv_ref[...]
    sfull = jnp.concatenate([s0[...], s1[...], s2[...], s3[...]], axis=1)
    gfull = jnp.concatenate([g0[...], g1[...], g2[...], g3[...]], axis=1)
    h = dinv * (sfull + gfull) + b_ref[...]
    h = jnp.maximum(h, 0.0)
    t = jnp.dot(h, w_ref[...], preferred_element_type=jnp.float32)
    t = t * dinv
    if split_out:
        for p in range(NP):
            outs[p][...] = t[:, p * W:(p + 1) * W]
    else:
        outs[0][...] = t


def _tc_mid(splanes, gplanes, dinv, b, Wm, split_out):
    kout = Wm.shape[1]
    if split_out:
        out_specs = _plane_specs()
        out_shape = [jax.ShapeDtypeStruct((N_PAD, W), jnp.float32)
                     for _ in range(NP)]
    else:
        out_specs = [pl.BlockSpec((_TB, kout), lambda i: (i, 0))]
        out_shape = [jax.ShapeDtypeStruct((N_PAD, kout), jnp.float32)]
    return pl.pallas_call(
        functools.partial(_t2_body, split_out=split_out),
        grid=(_TGRID,),
        in_specs=_plane_specs() + _plane_specs() + [
            pl.BlockSpec((_TB, 1), lambda i: (i, 0)),
            pl.BlockSpec((1, D), lambda i: (0, 0)),
            pl.BlockSpec((D, kout), lambda i: (0, 0)),
        ],
        out_specs=out_specs,
        out_shape=out_shape,
    )(*splanes, *gplanes, dinv, b, Wm)


def _t4_body(p_ref, g_ref, dinv_ref, b_ref, out_ref):
    sarr = jnp.sum(p_ref[...], axis=1, keepdims=True)
    dinv = dinv_ref[...]
    h = dinv * (sarr + g_ref[...]) + b_ref[...]
    out_ref[...] = jnp.maximum(h, 0.0)


def _tc_final(p, g, dinv, b3):
    return pl.pallas_call(
        _t4_body,
        grid=(_TGRID,),
        in_specs=[
            pl.BlockSpec((_TB, NW), lambda i: (i, 0)),
            pl.BlockSpec((_TB, 1), lambda i: (i, 0)),
            pl.BlockSpec((_TB, 1), lambda i: (i, 0)),
            pl.BlockSpec((1, 1), lambda i: (0, 0)),
        ],
        out_specs=pl.BlockSpec((_TB, 1), lambda i: (i, 0)),
        out_shape=jax.ShapeDtypeStruct((N_PAD, 1), jnp.float32),
    )(p, g, dinv, b3)


@jax.jit
def kernel(x, edge_index, edge_attr, W1, b1, W2, b2, W3, b3):
    src = edge_index[0]
    dst = edge_index[1]
    ew = jnp.squeeze(edge_attr)

    pad = E_PAD - E
    src_p = jnp.concatenate([src, jnp.zeros((pad,), jnp.int32)])
    dst_p = jnp.concatenate([dst, jnp.zeros((pad,), jnp.int32)])
    ew_p = jnp.concatenate([ew, jnp.zeros((pad,), jnp.float32)])
    src2 = src_p.reshape(NW * NCHUNK, CHUNK)
    dst2 = dst_p.reshape(NW * NCHUNK, CHUNK)
    ew2 = ew_p.reshape(NW * NCHUNK, CHUNK)

    x_pad = jnp.pad(x, ((0, N_PAD - N), (0, 0)))

    psrc, pdst, pew, pcnt = _part_kernel(src2, dst2, ew2)

    degp = _deg_kernel(dst2, ew2).reshape(NW, N_PAD).T

    t1 = _tc_first(degp, x_pad, W1)
    dinv, g1p = t1[0], t1[1:]

    s1 = _prop_kernel(psrc, pdst, pew, pcnt, *g1p).reshape(NP, N_PAD, W)

    g2p = _tc_mid(list(s1), list(g1p), dinv, b1.reshape(1, D), W2, True)

    s2 = _prop_kernel(psrc, pdst, pew, pcnt, *g2p).reshape(NP, N_PAD, W)

    (g3,) = _tc_mid(list(s2), list(g2p), dinv, b2.reshape(1, D), W3, False)

    s3 = _prop1_kernel(src2, dst2, ew2, g3[:, 0]).reshape(NW, N_PAD).T

    out = _tc_final(s3, g3, dinv, b3.reshape(1, 1))
    return jnp.squeeze(out[:N])
